# Initial kernel scaffold; baseline (speedup 1.0000x reference)
import jax
import jax.numpy as jnp
from jax.experimental import pallas as pl


def kernel(x, edge_index, edge_attr, batch, params):
    raise NotImplementedError("write your pallas kernel here")


# speedup vs baseline: 3.1704x; 3.1704x over previous
import functools

import jax
import jax.numpy as jnp
import numpy as np
from jax import lax
from jax.experimental import pallas as pl
from jax.experimental.pallas import tpu as pltpu
from jax.experimental.pallas import tpu_sc as plsc

N = 10000
E = 160000
D = 128
NG = 64
NC = 2
NS = 16
f32 = jnp.float32

_mesh = functools.partial(
    plsc.VectorSubcoreMesh, core_axis_name="c", subcore_axis_name="s")


def _sc_gather(table, idx, C, CH):
    R = idx.shape[0]
    rpw = R // (NC * NS)
    nch = rpw // CH

    @functools.partial(
        pl.kernel, mesh=_mesh(),
        out_type=jax.ShapeDtypeStruct((R, C), f32),
        scratch_types=[
            pltpu.VMEM((CH,), jnp.int32),
            pltpu.VMEM((CH, C), f32),
            pltpu.SemaphoreType.DMA,
        ])
    def k(table_hbm, idx_hbm, out_hbm, idx_v, rows_v, sem):
        wid = lax.axis_index("s") * NC + lax.axis_index("c")
        row0 = wid * rpw

        def body(i, carry):
            base = pl.multiple_of(row0 + i * CH, 8)
            pltpu.sync_copy(idx_hbm.at[pl.ds(base, CH)], idx_v)
            pltpu.async_copy(table_hbm.at[idx_v], rows_v, sem).wait()
            pltpu.sync_copy(rows_v, out_hbm.at[pl.ds(base, CH)])
            return carry

        lax.fori_loop(0, nch, body, 0)

    return k(table, idx)


def _sc_scatter(vals, idx, CH):
    R = vals.shape[0]
    rpt = R // (NC * NS)
    nch = rpt // CH
    NP = 10240
    nz = NP // NS

    @functools.partial(
        pl.kernel, mesh=_mesh(),
        out_type=jax.ShapeDtypeStruct((NC, NP, D), f32),
        scratch_types=[
            pltpu.VMEM((CH,), jnp.int32),
            pltpu.VMEM((CH, D), f32),
            pltpu.VMEM_SHARED((NP, D), f32),
            pltpu.SemaphoreType.DMA,
        ])
    def k(vals_hbm, idx_hbm, zero_hbm, out_hbm, idx_v, vals_v, acc, sem):
        c = lax.axis_index("c")
        s = lax.axis_index("s")
        pltpu.sync_copy(zero_hbm.at[pl.ds(s * nz, nz)],
                        acc.at[pl.ds(s * nz, nz)])
        plsc.subcore_barrier()
        row0 = c * (R // 2) + s * rpt

        def body(i, carry):
            base = pl.multiple_of(row0 + i * CH, 8)
            pltpu.sync_copy(idx_hbm.at[pl.ds(base, CH)], idx_v)
            pltpu.sync_copy(vals_hbm.at[pl.ds(base, CH)], vals_v)
            pltpu.sync_copy(vals_v, acc.at[idx_v], add=True)
            return carry

        lax.fori_loop(0, nch, body, 0)
        plsc.subcore_barrier()
        pltpu.sync_copy(acc.at[pl.ds(s * nz, nz)],
                        out_hbm.at[c, pl.ds(s * nz, nz)])

    return k(vals, idx, jnp.zeros((NP, D), f32))[:, :N, :]


def _silu(x):
    return x * jax.nn.sigmoid(x)


def _dot(a, b):
    return jnp.dot(a, b, preferred_element_type=f32)


def _rowln(x, g, b):
    m = jnp.mean(x, axis=-1, keepdims=True)
    v = jnp.mean((x - m) * (x - m), axis=-1, keepdims=True)
    return g * (x - m) / jnp.sqrt(v + 1e-5) + b


def _gn(xc, oh, w, b, ms):
    dn = (((0,), (0,)), ((), ()))
    cnt = jnp.sum(oh, axis=0)[:, None] + 1e-6
    mean = lax.dot_general(oh, xc, dn, preferred_element_type=f32) / cnt
    out = xc - _dot(oh, mean) * ms
    var = lax.dot_general(oh, out * out, dn, preferred_element_type=f32) / cnt
    out = out / jnp.sqrt(_dot(oh, var) + 1e-5)
    return out * w + b


def _onehot(b2d):
    return (b2d == lax.broadcasted_iota(jnp.int32, (N, NG), 1)).astype(f32)


_FULL = lambda shape: pl.BlockSpec(shape, lambda *_: tuple(0 for _ in shape))


BE = 2000
GE = E // BE


def _n1_body(x_r, wd_r, ws_r, t1_r):
    x = x_r[...]
    t1_r[...] = jnp.stack([_dot(x, wd_r[...]), _dot(x, ws_r[...])], axis=0)


def _edge1_body(gd_r, gs_r, ea_r, w1e_r, b1_r, wg2_r, bg2_r, wa2_r, ba2_r,
                g_o, m_o, st_o):
    pre = gd_r[...] + gs_r[...] + _dot(ea_r[...], w1e_r[...]) + b1_r[...]
    g = _dot(_silu(pre[:, :D]), wg2_r[...]) + bg2_r[...]
    m = _dot(_silu(pre[:, D:]), wa2_r[...]) + ba2_r[...]
    g_o[...] = g
    m_o[...] = m
    su = jnp.concatenate(
        [jnp.sum(g, 0, keepdims=True), jnp.sum(g * g, 0, keepdims=True),
         jnp.zeros((6, D), f32)], axis=0)

    @pl.when(pl.program_id(0) == 0)
    def _():
        st_o[...] = su

    @pl.when(pl.program_id(0) > 0)
    def _():
        st_o[...] = st_o[...] + su


def _edge2_body(g_r, m_r, st_r, bg_r, bb_r, sm_o):
    st = st_r[...]
    mean = st[0:1, :] / E
    var = st[1:2, :] / E - mean * mean
    sig = jax.nn.sigmoid(
        bg_r[...] * (g_r[...] - mean) / jnp.sqrt(var + 1e-5) + bb_r[...])
    sm_o[...] = jnp.stack([sig * m_r[...], sig], axis=0)


def _n2_body(x_r, num_r, den_r, b_r, bnxg_r, bnxb_r, gw_r, gb_r, gms_r,
             wq_r, bq_r, wk_r, bk_r, wv_r, bv_r, wsk_r, bsk_r,
             h_o, t2_o, xr_o):
    x = x_r[...]
    agg = num_r[...] / (den_r[...] + 1e-6)
    mu = jnp.mean(agg, 0, keepdims=True)
    va = jnp.mean((agg - mu) * (agg - mu), 0, keepdims=True)
    bn = bnxg_r[...] * (agg - mu) / jnp.sqrt(va + 1e-5) + bnxb_r[...]
    xcart = x + _silu(bn)
    oh = _onehot(b_r[...])
    h = x + _gn(xcart, oh, gw_r[...], gb_r[...], gms_r[...])
    h_o[...] = h
    q = _dot(h, wq_r[...]) + bq_r[...]
    kk = _dot(h, wk_r[...]) + bk_r[...]
    v = _dot(h, wv_r[...]) + bv_r[...]
    t2_o[...] = jnp.concatenate([q, kk, v], axis=1)
    xr_o[...] = _dot(h, wsk_r[...]) + bsk_r[...]


def _edge3_body(gd_r, gs_r, ea_r, we_r, wmu1_r, wmu2_r, wmu3_r, bmu_r,
                lag_r, lab_r, wml_r, bml_r, lmg_r, lmb_r, msg_o):
    gd = gd_r[...]
    gs = gs_r[...]
    q_i = gd[:, :D]
    k_i = gd[:, D:2 * D]
    v_i = gd[:, 2 * D:]
    k_j = gs[:, D:2 * D]
    v_j = gs[:, 2 * D:]
    e = _dot(ea_r[...], we_r[...])
    alpha = jnp.concatenate([q_i * k_i, q_i * k_j, q_i * e], axis=1)
    alpha = alpha * np.float32(1.0 / np.sqrt(3.0 * D))
    gate = jax.nn.sigmoid(_rowln(alpha, lag_r[...], lab_r[...]))
    msg0 = (_dot(v_i, wmu1_r[...]) + _dot(v_j, wmu2_r[...]) +
            _dot(e, wmu3_r[...]) + bmu_r[...])
    msg1 = _dot(msg0 * gate, wml_r[...]) + bml_r[...]
    msg_o[...] = _rowln(msg1, lmg_r[...], lmb_r[...])


def _n4_body(h_r, p_r, xr_r, b_r, wbo_r, wbx_r, gw_r, gb_r, gms_r, out_o):
    h = h_r[...]
    out = p_r[0] + p_r[1]
    xr = xr_r[...]
    beta = jax.nn.sigmoid(_dot(out, wbo_r[...]) + _dot(xr, wbx_r[...]))
    h_mat = beta * xr + (1.0 - beta) * out
    oh = _onehot(b_r[...])
    out_o[...] = h + _gn(h_mat, oh, gw_r[...], gb_r[...], gms_r[...])


def kernel(x, edge_index, edge_attr, batch, params):
    p = params
    src = edge_index[0]
    dst = edge_index[1]
    b2d = batch.reshape(N, 1)

    row2 = lambda a: a.reshape(1, -1)

    wd1 = jnp.concatenate([p['cart_Wg1'][:D], p['cart_Wa1'][:D]], axis=1)
    ws1 = jnp.concatenate([p['cart_Wg1'][D:2 * D], p['cart_Wa1'][D:2 * D]],
                          axis=1)
    w1e = jnp.concatenate([p['cart_Wg1'][2 * D:], p['cart_Wa1'][2 * D:]],
                          axis=1)
    b1 = row2(jnp.concatenate([p['cart_bg1'], p['cart_ba1']]))
    wbo = p['mat_Wbeta'][:D] + p['mat_Wbeta'][2 * D:]
    wbx = p['mat_Wbeta'][D:2 * D] - p['mat_Wbeta'][2 * D:]

    t1 = pl.pallas_call(
        _n1_body,
        grid=(),
        in_specs=[_FULL((N, D)), _FULL((D, 2 * D)), _FULL((D, 2 * D))],
        out_specs=_FULL((2, N, 2 * D)),
        out_shape=jax.ShapeDtypeStruct((2, N, 2 * D), f32),
    )(x, wd1, ws1)

    idx1 = jnp.concatenate([dst, src + N])
    g1 = _sc_gather(t1.reshape(2 * N, 2 * D), idx1, 2 * D, 400)

    espec = lambda w: pl.BlockSpec((BE, w), lambda i: (i, 0))
    espec_hi = lambda w: pl.BlockSpec((BE, w), lambda i: (GE + i, 0))
    g_arr, m_arr, st = pl.pallas_call(
        _edge1_body,
        grid=(GE,),
        in_specs=[espec(2 * D), espec_hi(2 * D), espec(D),
                  _FULL((D, 2 * D)), _FULL((1, 2 * D)),
                  _FULL((D, D)), _FULL((1, D)),
                  _FULL((D, D)), _FULL((1, D))],
        out_specs=[espec(D), espec(D),
                   pl.BlockSpec((8, D), lambda i: (0, 0))],
        out_shape=[jax.ShapeDtypeStruct((E, D), f32),
                   jax.ShapeDtypeStruct((E, D), f32),
                   jax.ShapeDtypeStruct((8, D), f32)],
    )(g1, g1, edge_attr, w1e, b1,
      p['cart_Wg2'], row2(p['cart_bg2']), p['cart_Wa2'], row2(p['cart_ba2']))

    sm = pl.pallas_call(
        _edge2_body,
        grid=(GE,),
        in_specs=[espec(D), espec(D), _FULL((8, D)),
                  _FULL((1, D)), _FULL((1, D))],
        out_specs=pl.BlockSpec((2, BE, D), lambda i: (0, i, 0)),
        out_shape=jax.ShapeDtypeStruct((2, E, D), f32),
    )(g_arr, m_arr, st, row2(p['cart_bne_g']), row2(p['cart_bne_b']))

    nd = _sc_scatter(sm.reshape(2 * E, D), jnp.concatenate([dst, dst]), 200)

    h, t2, xr = pl.pallas_call(
        _n2_body,
        grid=(),
        in_specs=[_FULL((N, D)), _FULL((N, D)), _FULL((N, D)),
                  _FULL((N, 1)),
                  _FULL((1, D)), _FULL((1, D)),
                  _FULL((1, D)), _FULL((1, D)), _FULL((1, D)),
                  _FULL((D, D)), _FULL((1, D)),
                  _FULL((D, D)), _FULL((1, D)),
                  _FULL((D, D)), _FULL((1, D)),
                  _FULL((D, D)), _FULL((1, D))],
        out_specs=[_FULL((N, D)), _FULL((N, 3 * D)), _FULL((N, D))],
        out_shape=[jax.ShapeDtypeStruct((N, D), f32),
                   jax.ShapeDtypeStruct((N, 3 * D), f32),
                   jax.ShapeDtypeStruct((N, D), f32)],
    )(x, nd[0], nd[1], b2d,
      row2(p['cart_bnx_g']), row2(p['cart_bnx_b']),
      row2(p['gnc_w']), row2(p['gnc_b']), row2(p['gnc_ms']),
      p['mat_Wq'], row2(p['mat_bq']), p['mat_Wk'], row2(p['mat_bk']),
      p['mat_Wv'], row2(p['mat_bv']), p['mat_Wskip'], row2(p['mat_bskip']))

    idx2 = jnp.concatenate([dst, src])
    g2 = _sc_gather(t2, idx2, 3 * D, 200)

    msg = pl.pallas_call(
        _edge3_body,
        grid=(GE,),
        in_specs=[espec(3 * D), espec_hi(3 * D), espec(D),
                  _FULL((D, D)),
                  _FULL((D, 3 * D)), _FULL((D, 3 * D)), _FULL((D, 3 * D)),
                  _FULL((1, 3 * D)), _FULL((1, 3 * D)), _FULL((1, 3 * D)),
                  _FULL((3 * D, D)), _FULL((1, D)),
                  _FULL((1, D)), _FULL((1, D))],
        out_specs=espec(D),
        out_shape=jax.ShapeDtypeStruct((E, D), f32),
    )(g2, g2, edge_attr, p['mat_We'],
      p['mat_Wmu'][:D], p['mat_Wmu'][D:2 * D], p['mat_Wmu'][2 * D:],
      row2(p['mat_bmu']), row2(p['mat_lna_g']), row2(p['mat_lna_b']),
      p['mat_Wml'], row2(p['mat_bml']),
      row2(p['mat_lnm_g']), row2(p['mat_lnm_b']))

    mo = _sc_scatter(msg, dst, 200)

    return pl.pallas_call(
        _n4_body,
        grid=(),
        in_specs=[_FULL((N, D)), _FULL((2, N, D)), _FULL((N, D)),
                  _FULL((N, 1)), _FULL((D, 1)), _FULL((D, 1)),
                  _FULL((1, D)), _FULL((1, D)), _FULL((1, D))],
        out_specs=_FULL((N, D)),
        out_shape=jax.ShapeDtypeStruct((N, D), f32),
    )(h, mo, xr, b2d, wbo, wbx,
      row2(p['gnm_w']), row2(p['gnm_b']), row2(p['gnm_ms']))


# speedup vs baseline: 3.1851x; 1.0046x over previous
import functools

import jax
import jax.numpy as jnp
import numpy as np
from jax import lax
from jax.experimental import pallas as pl
from jax.experimental.pallas import tpu as pltpu
from jax.experimental.pallas import tpu_sc as plsc

N = 10000
E = 160000
D = 128
NG = 64
NC = 2
NS = 16
f32 = jnp.float32

_mesh = functools.partial(
    plsc.VectorSubcoreMesh, core_axis_name="c", subcore_axis_name="s")


def _sc_gather(table, idx, C, CH):
    R = idx.shape[0]
    rpw = R // (NC * NS)
    nch = rpw // CH

    @functools.partial(
        pl.kernel, mesh=_mesh(),
        out_type=jax.ShapeDtypeStruct((R, C), f32),
        scratch_types=[
            pltpu.VMEM((CH,), jnp.int32),
            pltpu.VMEM((CH, C), f32),
            pltpu.SemaphoreType.DMA,
        ])
    def k(table_hbm, idx_hbm, out_hbm, idx_v, rows_v, sem):
        wid = lax.axis_index("s") * NC + lax.axis_index("c")
        row0 = wid * rpw

        def body(i, carry):
            base = pl.multiple_of(row0 + i * CH, 8)
            pltpu.sync_copy(idx_hbm.at[pl.ds(base, CH)], idx_v)
            pltpu.async_copy(table_hbm.at[idx_v], rows_v, sem).wait()
            pltpu.sync_copy(rows_v, out_hbm.at[pl.ds(base, CH)])
            return carry

        lax.fori_loop(0, nch, body, 0)

    return k(table, idx)


def _sc_scatter(vals, idx, CH):
    R = vals.shape[0]
    rpt = R // (NC * NS)
    nch = rpt // CH
    NP = 10240
    nz = NP // NS

    @functools.partial(
        pl.kernel, mesh=_mesh(),
        out_type=jax.ShapeDtypeStruct((NC, NP, D), f32),
        scratch_types=[
            pltpu.VMEM((CH,), jnp.int32),
            pltpu.VMEM((CH, D), f32),
            pltpu.VMEM_SHARED((NP, D), f32),
            pltpu.SemaphoreType.DMA,
        ])
    def k(vals_hbm, idx_hbm, zero_hbm, out_hbm, idx_v, vals_v, acc, sem):
        c = lax.axis_index("c")
        s = lax.axis_index("s")
        pltpu.sync_copy(zero_hbm.at[pl.ds(s * nz, nz)],
                        acc.at[pl.ds(s * nz, nz)])
        plsc.subcore_barrier()
        row0 = c * (R // 2) + s * rpt

        def body(i, carry):
            base = pl.multiple_of(row0 + i * CH, 8)
            pltpu.sync_copy(idx_hbm.at[pl.ds(base, CH)], idx_v)
            pltpu.sync_copy(vals_hbm.at[pl.ds(base, CH)], vals_v)
            pltpu.sync_copy(vals_v, acc.at[idx_v], add=True)
            return carry

        lax.fori_loop(0, nch, body, 0)
        plsc.subcore_barrier()
        pltpu.sync_copy(acc.at[pl.ds(s * nz, nz)],
                        out_hbm.at[c, pl.ds(s * nz, nz)])

    return k(vals, idx, jnp.zeros((NP, D), f32))[:, :N, :]


def _silu(x):
    return x * jax.nn.sigmoid(x)


def _dot(a, b):
    return jnp.dot(a, b, preferred_element_type=f32)


def _bdot(a, b):
    return jnp.dot(a.astype(jnp.bfloat16), b.astype(jnp.bfloat16),
                   preferred_element_type=f32)


def _rowln(x, g, b):
    m = jnp.mean(x, axis=-1, keepdims=True)
    v = jnp.mean((x - m) * (x - m), axis=-1, keepdims=True)
    return g * (x - m) / jnp.sqrt(v + 1e-5) + b


def _gn(xc, oh, w, b, ms):
    dn = (((0,), (0,)), ((), ()))
    cnt = jnp.sum(oh, axis=0)[:, None] + 1e-6
    mean = lax.dot_general(oh, xc, dn, preferred_element_type=f32) / cnt
    out = xc - _dot(oh, mean) * ms
    var = lax.dot_general(oh, out * out, dn, preferred_element_type=f32) / cnt
    out = out / jnp.sqrt(_dot(oh, var) + 1e-5)
    return out * w + b


def _onehot(b2d):
    return (b2d == lax.broadcasted_iota(jnp.int32, (N, NG), 1)).astype(f32)


_FULL = lambda shape: pl.BlockSpec(shape, lambda *_: tuple(0 for _ in shape))


BE = 2000
GE = E // BE


def _n1_body(x_r, wd_r, ws_r, t1_r):
    x = x_r[...]
    t1_r[...] = jnp.stack([_dot(x, wd_r[...]), _dot(x, ws_r[...])], axis=0)


def _edge1_body(gd_r, gs_r, ea_r, w1e_r, b1_r, wg2_r, bg2_r, wa2_r, ba2_r,
                g_o, m_o, st_o):
    pre = gd_r[...] + gs_r[...] + _bdot(ea_r[...], w1e_r[...]) + b1_r[...]
    g = _bdot(_silu(pre[:, :D]), wg2_r[...]) + bg2_r[...]
    m = _bdot(_silu(pre[:, D:]), wa2_r[...]) + ba2_r[...]
    g_o[...] = g.astype(jnp.bfloat16)
    m_o[...] = m.astype(jnp.bfloat16)
    su = jnp.concatenate(
        [jnp.sum(g, 0, keepdims=True), jnp.sum(g * g, 0, keepdims=True),
         jnp.zeros((6, D), f32)], axis=0)

    @pl.when(pl.program_id(0) == 0)
    def _():
        st_o[...] = su

    @pl.when(pl.program_id(0) > 0)
    def _():
        st_o[...] = st_o[...] + su


def _edge2_body(g_r, m_r, st_r, bg_r, bb_r, sm_o):
    st = st_r[...]
    mean = st[0:1, :] / E
    var = st[1:2, :] / E - mean * mean
    sig = jax.nn.sigmoid(
        bg_r[...] * (g_r[...].astype(f32) - mean) / jnp.sqrt(var + 1e-5)
        + bb_r[...])
    sm_o[...] = jnp.stack([sig * m_r[...].astype(f32), sig], axis=0)


def _n2_body(x_r, num_r, den_r, b_r, bnxg_r, bnxb_r, gw_r, gb_r, gms_r,
             wq_r, bq_r, wk_r, bk_r, wv_r, bv_r, wsk_r, bsk_r,
             h_o, t2_o, xr_o):
    x = x_r[...]
    agg = num_r[...] / (den_r[...] + 1e-6)
    mu = jnp.mean(agg, 0, keepdims=True)
    va = jnp.mean((agg - mu) * (agg - mu), 0, keepdims=True)
    bn = bnxg_r[...] * (agg - mu) / jnp.sqrt(va + 1e-5) + bnxb_r[...]
    xcart = x + _silu(bn)
    oh = _onehot(b_r[...])
    h = x + _gn(xcart, oh, gw_r[...], gb_r[...], gms_r[...])
    h_o[...] = h
    q = _dot(h, wq_r[...]) + bq_r[...]
    kk = _dot(h, wk_r[...]) + bk_r[...]
    v = _dot(h, wv_r[...]) + bv_r[...]
    t2_o[...] = jnp.concatenate([q, kk, v], axis=1)
    xr_o[...] = _dot(h, wsk_r[...]) + bsk_r[...]


def _edge3_body(gd_r, gs_r, ea_r, we_r, wmu1_r, wmu2_r, wmu3_r, bmu_r,
                lag_r, lab_r, wml_r, bml_r, lmg_r, lmb_r, msg_o):
    gd = gd_r[...]
    gs = gs_r[...]
    q_i = gd[:, :D]
    k_i = gd[:, D:2 * D]
    v_i = gd[:, 2 * D:]
    k_j = gs[:, D:2 * D]
    v_j = gs[:, 2 * D:]
    e = _bdot(ea_r[...], we_r[...])
    alpha = jnp.concatenate([q_i * k_i, q_i * k_j, q_i * e], axis=1)
    alpha = alpha * np.float32(1.0 / np.sqrt(3.0 * D))
    gate = jax.nn.sigmoid(_rowln(alpha, lag_r[...], lab_r[...]))
    msg0 = (_bdot(v_i, wmu1_r[...]) + _bdot(v_j, wmu2_r[...]) +
            _bdot(e, wmu3_r[...]) + bmu_r[...])
    msg1 = _bdot(msg0 * gate, wml_r[...]) + bml_r[...]
    msg_o[...] = _rowln(msg1, lmg_r[...], lmb_r[...])


def _n4_body(h_r, p_r, xr_r, b_r, wbo_r, wbx_r, gw_r, gb_r, gms_r, out_o):
    h = h_r[...]
    out = p_r[0] + p_r[1]
    xr = xr_r[...]
    beta = jax.nn.sigmoid(_dot(out, wbo_r[...]) + _dot(xr, wbx_r[...]))
    h_mat = beta * xr + (1.0 - beta) * out
    oh = _onehot(b_r[...])
    out_o[...] = h + _gn(h_mat, oh, gw_r[...], gb_r[...], gms_r[...])


def kernel(x, edge_index, edge_attr, batch, params):
    p = params
    src = edge_index[0]
    dst = edge_index[1]
    b2d = batch.reshape(N, 1)

    row2 = lambda a: a.reshape(1, -1)

    wd1 = jnp.concatenate([p['cart_Wg1'][:D], p['cart_Wa1'][:D]], axis=1)
    ws1 = jnp.concatenate([p['cart_Wg1'][D:2 * D], p['cart_Wa1'][D:2 * D]],
                          axis=1)
    w1e = jnp.concatenate([p['cart_Wg1'][2 * D:], p['cart_Wa1'][2 * D:]],
                          axis=1)
    b1 = row2(jnp.concatenate([p['cart_bg1'], p['cart_ba1']]))
    wbo = p['mat_Wbeta'][:D] + p['mat_Wbeta'][2 * D:]
    wbx = p['mat_Wbeta'][D:2 * D] - p['mat_Wbeta'][2 * D:]

    t1 = pl.pallas_call(
        _n1_body,
        grid=(),
        in_specs=[_FULL((N, D)), _FULL((D, 2 * D)), _FULL((D, 2 * D))],
        out_specs=_FULL((2, N, 2 * D)),
        out_shape=jax.ShapeDtypeStruct((2, N, 2 * D), f32),
    )(x, wd1, ws1)

    idx1 = jnp.concatenate([dst, src + N])
    g1 = _sc_gather(t1.reshape(2 * N, 2 * D), idx1, 2 * D, 400)

    espec = lambda w: pl.BlockSpec((BE, w), lambda i: (i, 0))
    espec_hi = lambda w: pl.BlockSpec((BE, w), lambda i: (GE + i, 0))
    g_arr, m_arr, st = pl.pallas_call(
        _edge1_body,
        grid=(GE,),
        in_specs=[espec(2 * D), espec_hi(2 * D), espec(D),
                  _FULL((D, 2 * D)), _FULL((1, 2 * D)),
                  _FULL((D, D)), _FULL((1, D)),
                  _FULL((D, D)), _FULL((1, D))],
        out_specs=[espec(D), espec(D),
                   pl.BlockSpec((8, D), lambda i: (0, 0))],
        out_shape=[jax.ShapeDtypeStruct((E, D), jnp.bfloat16),
                   jax.ShapeDtypeStruct((E, D), jnp.bfloat16),
                   jax.ShapeDtypeStruct((8, D), f32)],
    )(g1, g1, edge_attr, w1e, b1,
      p['cart_Wg2'], row2(p['cart_bg2']), p['cart_Wa2'], row2(p['cart_ba2']))

    sm = pl.pallas_call(
        _edge2_body,
        grid=(GE,),
        in_specs=[espec(D), espec(D), _FULL((8, D)),
                  _FULL((1, D)), _FULL((1, D))],
        out_specs=pl.BlockSpec((2, BE, D), lambda i: (0, i, 0)),
        out_shape=jax.ShapeDtypeStruct((2, E, D), f32),
    )(g_arr, m_arr, st, row2(p['cart_bne_g']), row2(p['cart_bne_b']))

    nd = _sc_scatter(sm.reshape(2 * E, D), jnp.concatenate([dst, dst]), 200)

    h, t2, xr = pl.pallas_call(
        _n2_body,
        grid=(),
        in_specs=[_FULL((N, D)), _FULL((N, D)), _FULL((N, D)),
                  _FULL((N, 1)),
                  _FULL((1, D)), _FULL((1, D)),
                  _FULL((1, D)), _FULL((1, D)), _FULL((1, D)),
                  _FULL((D, D)), _FULL((1, D)),
                  _FULL((D, D)), _FULL((1, D)),
                  _FULL((D, D)), _FULL((1, D)),
                  _FULL((D, D)), _FULL((1, D))],
        out_specs=[_FULL((N, D)), _FULL((N, 3 * D)), _FULL((N, D))],
        out_shape=[jax.ShapeDtypeStruct((N, D), f32),
                   jax.ShapeDtypeStruct((N, 3 * D), f32),
                   jax.ShapeDtypeStruct((N, D), f32)],
    )(x, nd[0], nd[1], b2d,
      row2(p['cart_bnx_g']), row2(p['cart_bnx_b']),
      row2(p['gnc_w']), row2(p['gnc_b']), row2(p['gnc_ms']),
      p['mat_Wq'], row2(p['mat_bq']), p['mat_Wk'], row2(p['mat_bk']),
      p['mat_Wv'], row2(p['mat_bv']), p['mat_Wskip'], row2(p['mat_bskip']))

    idx2 = jnp.concatenate([dst, src])
    g2 = _sc_gather(t2, idx2, 3 * D, 200)

    msg = pl.pallas_call(
        _edge3_body,
        grid=(GE,),
        in_specs=[espec(3 * D), espec_hi(3 * D), espec(D),
                  _FULL((D, D)),
                  _FULL((D, 3 * D)), _FULL((D, 3 * D)), _FULL((D, 3 * D)),
                  _FULL((1, 3 * D)), _FULL((1, 3 * D)), _FULL((1, 3 * D)),
                  _FULL((3 * D, D)), _FULL((1, D)),
                  _FULL((1, D)), _FULL((1, D))],
        out_specs=espec(D),
        out_shape=jax.ShapeDtypeStruct((E, D), f32),
    )(g2, g2, edge_attr, p['mat_We'],
      p['mat_Wmu'][:D], p['mat_Wmu'][D:2 * D], p['mat_Wmu'][2 * D:],
      row2(p['mat_bmu']), row2(p['mat_lna_g']), row2(p['mat_lna_b']),
      p['mat_Wml'], row2(p['mat_bml']),
      row2(p['mat_lnm_g']), row2(p['mat_lnm_b']))

    mo = _sc_scatter(msg, dst, 200)

    return pl.pallas_call(
        _n4_body,
        grid=(),
        in_specs=[_FULL((N, D)), _FULL((2, N, D)), _FULL((N, D)),
                  _FULL((N, 1)), _FULL((D, 1)), _FULL((D, 1)),
                  _FULL((1, D)), _FULL((1, D)), _FULL((1, D))],
        out_specs=_FULL((N, D)),
        out_shape=jax.ShapeDtypeStruct((N, D), f32),
    )(h, mo, xr, b2d, wbo, wbx,
      row2(p['gnm_w']), row2(p['gnm_b']), row2(p['gnm_ms']))


# speedup vs baseline: 3.2721x; 1.0273x over previous
import functools

import jax
import jax.numpy as jnp
import numpy as np
from jax import lax
from jax.experimental import pallas as pl
from jax.experimental.pallas import tpu as pltpu
from jax.experimental.pallas import tpu_sc as plsc

N = 10000
E = 160000
D = 128
NG = 64
NC = 2
NS = 16
f32 = jnp.float32

_mesh = functools.partial(
    plsc.VectorSubcoreMesh, core_axis_name="c", subcore_axis_name="s")


NP = 10240


def _sc_gather(table, idx, C, CH):
    R = idx.shape[0]
    rpw = R // (NC * NS)
    nch = rpw // CH
    npairs = nch // 2
    tail = nch % 2 == 1

    @functools.partial(
        pl.kernel, mesh=_mesh(),
        out_type=jax.ShapeDtypeStruct((R, C), f32),
        scratch_types=[
            pltpu.VMEM((CH,), jnp.int32),
            pltpu.VMEM((CH,), jnp.int32),
            pltpu.VMEM((CH, C), f32),
            pltpu.VMEM((CH, C), f32),
            pltpu.SemaphoreType.DMA,
            pltpu.SemaphoreType.DMA,
            pltpu.SemaphoreType.DMA,
            pltpu.SemaphoreType.DMA,
        ])
    def k(table_hbm, idx_hbm, out_hbm, i0, i1, r0, r1, g0, g1, w0, w1):
        wid = lax.axis_index("s") * NC + lax.axis_index("c")
        row0 = wid * rpw
        idx_v = (i0, i1)
        rows_v = (r0, r1)
        gsem = (g0, g1)
        wsem = (w0, w1)

        def pair(j, carry):
            hs = []
            for b in range(2):
                base = pl.multiple_of(row0 + (2 * j + b) * CH, 8)

                @pl.when(j > 0)
                def _():
                    pltpu.make_async_copy(
                        out_hbm.at[pl.ds(0, CH)], rows_v[b], wsem[b]).wait()

                pltpu.sync_copy(idx_hbm.at[pl.ds(base, CH)], idx_v[b])
                hs.append(pltpu.async_copy(table_hbm.at[idx_v[b]], rows_v[b],
                                           gsem[b]))
            for b in range(2):
                base = pl.multiple_of(row0 + (2 * j + b) * CH, 8)
                hs[b].wait()
                pltpu.async_copy(rows_v[b], out_hbm.at[pl.ds(base, CH)],
                                 wsem[b])
            return carry

        lax.fori_loop(0, npairs, pair, 0)
        for b in range(2):
            pltpu.make_async_copy(
                out_hbm.at[pl.ds(0, CH)], rows_v[b], wsem[b]).wait()
        if tail:
            base = pl.multiple_of(row0 + 2 * npairs * CH, 8)
            pltpu.sync_copy(idx_hbm.at[pl.ds(base, CH)], i0)
            pltpu.async_copy(table_hbm.at[i0], r0, g0).wait()
            pltpu.sync_copy(r0, out_hbm.at[pl.ds(base, CH)])

    return k(table, idx)


def _sc_scatter(vals, idx, CH):
    R = vals.shape[0]
    rpt = R // (NC * NS)
    nch = rpt // CH
    nz = NP // NS

    @functools.partial(
        pl.kernel, mesh=_mesh(),
        out_type=jax.ShapeDtypeStruct((NC, NP, D), f32),
        scratch_types=[
            pltpu.VMEM((CH,), jnp.int32),
            pltpu.VMEM((CH, D), f32),
            pltpu.VMEM_SHARED((NP, D), f32),
            pltpu.SemaphoreType.DMA,
        ])
    def k(vals_hbm, idx_hbm, zero_hbm, out_hbm, idx_v, vals_v, acc, sem):
        c = lax.axis_index("c")
        s = lax.axis_index("s")
        pltpu.sync_copy(zero_hbm.at[pl.ds(s * nz, nz)],
                        acc.at[pl.ds(s * nz, nz)])
        plsc.subcore_barrier()
        row0 = c * (R // 2) + s * rpt

        def body(i, carry):
            base = pl.multiple_of(row0 + i * CH, 8)
            pltpu.sync_copy(idx_hbm.at[pl.ds(base, CH)], idx_v)
            pltpu.sync_copy(vals_hbm.at[pl.ds(base, CH)], vals_v)
            pltpu.sync_copy(vals_v, acc.at[idx_v], add=True)
            return carry

        lax.fori_loop(0, nch, body, 0)
        plsc.subcore_barrier()
        pltpu.sync_copy(acc.at[pl.ds(s * nz, nz)],
                        out_hbm.at[c, pl.ds(s * nz, nz)])

    return k(vals, idx, jnp.zeros((NP, D), f32))[:, :N, :]


def _silu(x):
    return x * jax.nn.sigmoid(x)


def _dot(a, b):
    return jnp.dot(a, b, preferred_element_type=f32)


def _bdot(a, b):
    return jnp.dot(a.astype(jnp.bfloat16), b.astype(jnp.bfloat16),
                   preferred_element_type=f32)


def _rowln(x, g, b):
    m = jnp.mean(x, axis=-1, keepdims=True)
    v = jnp.mean((x - m) * (x - m), axis=-1, keepdims=True)
    return g * (x - m) / jnp.sqrt(v + 1e-5) + b


def _gn(xc, oh, w, b, ms):
    dn = (((0,), (0,)), ((), ()))
    cnt = jnp.sum(oh, axis=0)[:, None] + 1e-6
    mean = lax.dot_general(oh, xc, dn, preferred_element_type=f32) / cnt
    out = xc - _dot(oh, mean) * ms
    var = lax.dot_general(oh, out * out, dn, preferred_element_type=f32) / cnt
    out = out / jnp.sqrt(_dot(oh, var) + 1e-5)
    return out * w + b


def _onehot(b2d):
    return (b2d == lax.broadcasted_iota(jnp.int32, (N, NG), 1)).astype(f32)


_FULL = lambda shape: pl.BlockSpec(shape, lambda *_: tuple(0 for _ in shape))


BE = 2000
GE = E // BE


def _n1_body(x_r, wd_r, ws_r, t1_r):
    x = x_r[...]
    t1_r[...] = jnp.stack([_dot(x, wd_r[...]), _dot(x, ws_r[...])], axis=0)


def _edge1_body(gd_r, gs_r, ea_r, w1e_r, b1_r, wg2_r, bg2_r, wa2_r, ba2_r,
                g_o, m_o, st_o):
    pre = gd_r[...] + gs_r[...] + _bdot(ea_r[...], w1e_r[...]) + b1_r[...]
    g = _bdot(_silu(pre[:, :D]), wg2_r[...]) + bg2_r[...]
    m = _bdot(_silu(pre[:, D:]), wa2_r[...]) + ba2_r[...]
    g_o[...] = g.astype(jnp.bfloat16)
    m_o[...] = m.astype(jnp.bfloat16)
    su = jnp.concatenate(
        [jnp.sum(g, 0, keepdims=True), jnp.sum(g * g, 0, keepdims=True),
         jnp.zeros((6, D), f32)], axis=0)

    @pl.when(pl.program_id(0) == 0)
    def _():
        st_o[...] = su

    @pl.when(pl.program_id(0) > 0)
    def _():
        st_o[...] = st_o[...] + su


def _edge2_body(g_r, m_r, st_r, bg_r, bb_r, sm_o):
    st = st_r[...]
    mean = st[0:1, :] / E
    var = st[1:2, :] / E - mean * mean
    sig = jax.nn.sigmoid(
        bg_r[...] * (g_r[...].astype(f32) - mean) / jnp.sqrt(var + 1e-5)
        + bb_r[...])
    sm_o[...] = jnp.stack([sig * m_r[...].astype(f32), sig], axis=0)


def _n2_body(x_r, num_r, den_r, b_r, bnxg_r, bnxb_r, gw_r, gb_r, gms_r,
             wq_r, bq_r, wk_r, bk_r, wv_r, bv_r, wsk_r, bsk_r,
             h_o, t2_o, xr_o):
    x = x_r[...]
    agg = num_r[...] / (den_r[...] + 1e-6)
    mu = jnp.mean(agg, 0, keepdims=True)
    va = jnp.mean((agg - mu) * (agg - mu), 0, keepdims=True)
    bn = bnxg_r[...] * (agg - mu) / jnp.sqrt(va + 1e-5) + bnxb_r[...]
    xcart = x + _silu(bn)
    oh = _onehot(b_r[...])
    h = x + _gn(xcart, oh, gw_r[...], gb_r[...], gms_r[...])
    h_o[...] = h
    q = _dot(h, wq_r[...]) + bq_r[...]
    kk = _dot(h, wk_r[...]) + bk_r[...]
    v = _dot(h, wv_r[...]) + bv_r[...]
    t2_o[...] = jnp.concatenate([q, kk, v], axis=1)
    xr_o[...] = _dot(h, wsk_r[...]) + bsk_r[...]


def _edge3_body(gd_r, gs_r, ea_r, we_r, wmu1_r, wmu2_r, wmu3_r, bmu_r,
                lag_r, lab_r, wml_r, bml_r, lmg_r, lmb_r, msg_o):
    gd = gd_r[...]
    gs = gs_r[...]
    q_i = gd[:, :D]
    k_i = gd[:, D:2 * D]
    v_i = gd[:, 2 * D:]
    k_j = gs[:, D:2 * D]
    v_j = gs[:, 2 * D:]
    e = _bdot(ea_r[...], we_r[...])
    alpha = jnp.concatenate([q_i * k_i, q_i * k_j, q_i * e], axis=1)
    alpha = alpha * np.float32(1.0 / np.sqrt(3.0 * D))
    gate = jax.nn.sigmoid(_rowln(alpha, lag_r[...], lab_r[...]))
    msg0 = (_bdot(v_i, wmu1_r[...]) + _bdot(v_j, wmu2_r[...]) +
            _bdot(e, wmu3_r[...]) + bmu_r[...])
    msg1 = _bdot(msg0 * gate, wml_r[...]) + bml_r[...]
    msg_o[...] = _rowln(msg1, lmg_r[...], lmb_r[...])


def _n4_body(h_r, p_r, xr_r, b_r, wbo_r, wbx_r, gw_r, gb_r, gms_r, out_o):
    h = h_r[...]
    out = p_r[0] + p_r[1]
    xr = xr_r[...]
    beta = jax.nn.sigmoid(_dot(out, wbo_r[...]) + _dot(xr, wbx_r[...]))
    h_mat = beta * xr + (1.0 - beta) * out
    oh = _onehot(b_r[...])
    out_o[...] = h + _gn(h_mat, oh, gw_r[...], gb_r[...], gms_r[...])


def kernel(x, edge_index, edge_attr, batch, params):
    p = params
    src = edge_index[0]
    dst = edge_index[1]
    b2d = batch.reshape(N, 1)

    row2 = lambda a: a.reshape(1, -1)

    wd1 = jnp.concatenate([p['cart_Wg1'][:D], p['cart_Wa1'][:D]], axis=1)
    ws1 = jnp.concatenate([p['cart_Wg1'][D:2 * D], p['cart_Wa1'][D:2 * D]],
                          axis=1)
    w1e = jnp.concatenate([p['cart_Wg1'][2 * D:], p['cart_Wa1'][2 * D:]],
                          axis=1)
    b1 = row2(jnp.concatenate([p['cart_bg1'], p['cart_ba1']]))
    wbo = p['mat_Wbeta'][:D] + p['mat_Wbeta'][2 * D:]
    wbx = p['mat_Wbeta'][D:2 * D] - p['mat_Wbeta'][2 * D:]

    t1 = pl.pallas_call(
        _n1_body,
        grid=(),
        in_specs=[_FULL((N, D)), _FULL((D, 2 * D)), _FULL((D, 2 * D))],
        out_specs=_FULL((2, N, 2 * D)),
        out_shape=jax.ShapeDtypeStruct((2, N, 2 * D), f32),
    )(x, wd1, ws1)

    idx1 = jnp.concatenate([dst, src + N])
    g1 = _sc_gather(t1.reshape(2 * N, 2 * D), idx1, 2 * D, 200)

    espec = lambda w: pl.BlockSpec((BE, w), lambda i: (i, 0))
    espec_hi = lambda w: pl.BlockSpec((BE, w), lambda i: (GE + i, 0))
    g_arr, m_arr, st = pl.pallas_call(
        _edge1_body,
        grid=(GE,),
        in_specs=[espec(2 * D), espec_hi(2 * D), espec(D),
                  _FULL((D, 2 * D)), _FULL((1, 2 * D)),
                  _FULL((D, D)), _FULL((1, D)),
                  _FULL((D, D)), _FULL((1, D))],
        out_specs=[espec(D), espec(D),
                   pl.BlockSpec((8, D), lambda i: (0, 0))],
        out_shape=[jax.ShapeDtypeStruct((E, D), jnp.bfloat16),
                   jax.ShapeDtypeStruct((E, D), jnp.bfloat16),
                   jax.ShapeDtypeStruct((8, D), f32)],
    )(g1, g1, edge_attr, w1e, b1,
      p['cart_Wg2'], row2(p['cart_bg2']), p['cart_Wa2'], row2(p['cart_ba2']))

    sm = pl.pallas_call(
        _edge2_body,
        grid=(GE,),
        in_specs=[espec(D), espec(D), _FULL((8, D)),
                  _FULL((1, D)), _FULL((1, D))],
        out_specs=pl.BlockSpec((2, BE, D), lambda i: (0, i, 0)),
        out_shape=jax.ShapeDtypeStruct((2, E, D), f32),
    )(g_arr, m_arr, st, row2(p['cart_bne_g']), row2(p['cart_bne_b']))

    nd = _sc_scatter(sm.reshape(2 * E, D), jnp.concatenate([dst, dst]), 200)
    num = nd[0]
    den = nd[1]

    h, t2, xr = pl.pallas_call(
        _n2_body,
        grid=(),
        in_specs=[_FULL((N, D)), _FULL((N, D)), _FULL((N, D)),
                  _FULL((N, 1)),
                  _FULL((1, D)), _FULL((1, D)),
                  _FULL((1, D)), _FULL((1, D)), _FULL((1, D)),
                  _FULL((D, D)), _FULL((1, D)),
                  _FULL((D, D)), _FULL((1, D)),
                  _FULL((D, D)), _FULL((1, D)),
                  _FULL((D, D)), _FULL((1, D))],
        out_specs=[_FULL((N, D)), _FULL((N, 3 * D)), _FULL((N, D))],
        out_shape=[jax.ShapeDtypeStruct((N, D), f32),
                   jax.ShapeDtypeStruct((N, 3 * D), f32),
                   jax.ShapeDtypeStruct((N, D), f32)],
    )(x, num, den, b2d,
      row2(p['cart_bnx_g']), row2(p['cart_bnx_b']),
      row2(p['gnc_w']), row2(p['gnc_b']), row2(p['gnc_ms']),
      p['mat_Wq'], row2(p['mat_bq']), p['mat_Wk'], row2(p['mat_bk']),
      p['mat_Wv'], row2(p['mat_bv']), p['mat_Wskip'], row2(p['mat_bskip']))

    idx2 = jnp.concatenate([dst, src])
    g2 = _sc_gather(t2, idx2, 3 * D, 80)

    msg = pl.pallas_call(
        _edge3_body,
        grid=(GE,),
        in_specs=[espec(3 * D), espec_hi(3 * D), espec(D),
                  _FULL((D, D)),
                  _FULL((D, 3 * D)), _FULL((D, 3 * D)), _FULL((D, 3 * D)),
                  _FULL((1, 3 * D)), _FULL((1, 3 * D)), _FULL((1, 3 * D)),
                  _FULL((3 * D, D)), _FULL((1, D)),
                  _FULL((1, D)), _FULL((1, D))],
        out_specs=espec(D),
        out_shape=jax.ShapeDtypeStruct((E, D), f32),
    )(g2, g2, edge_attr, p['mat_We'],
      p['mat_Wmu'][:D], p['mat_Wmu'][D:2 * D], p['mat_Wmu'][2 * D:],
      row2(p['mat_bmu']), row2(p['mat_lna_g']), row2(p['mat_lna_b']),
      p['mat_Wml'], row2(p['mat_bml']),
      row2(p['mat_lnm_g']), row2(p['mat_lnm_b']))

    mo = _sc_scatter(msg, dst, 200)

    return pl.pallas_call(
        _n4_body,
        grid=(),
        in_specs=[_FULL((N, D)), _FULL((2, N, D)), _FULL((N, D)),
                  _FULL((N, 1)), _FULL((D, 1)), _FULL((D, 1)),
                  _FULL((1, D)), _FULL((1, D)), _FULL((1, D))],
        out_specs=_FULL((N, D)),
        out_shape=jax.ShapeDtypeStruct((N, D), f32),
    )(h, mo, xr, b2d, wbo, wbx,
      row2(p['gnm_w']), row2(p['gnm_b']), row2(p['gnm_ms']))


# speedup vs baseline: 3.8932x; 1.1898x over previous
import functools

import jax
import jax.numpy as jnp
import numpy as np
from jax import lax
from jax.experimental import pallas as pl
from jax.experimental.pallas import tpu as pltpu
from jax.experimental.pallas import tpu_sc as plsc

N = 10000
E = 160000
D = 128
NG = 64
NC = 2
NS = 16
f32 = jnp.float32

_mesh = functools.partial(
    plsc.VectorSubcoreMesh, core_axis_name="c", subcore_axis_name="s")


NP = 10240


def _sc_gather(table, idx, C, CH):
    R = idx.shape[0]
    rpw = R // (NC * NS)
    nch = rpw // CH
    npairs = nch // 2
    tail = nch % 2 == 1

    @functools.partial(
        pl.kernel, mesh=_mesh(),
        out_type=jax.ShapeDtypeStruct((R, C), table.dtype),
        scratch_types=[
            pltpu.VMEM((CH,), jnp.int32),
            pltpu.VMEM((CH,), jnp.int32),
            pltpu.VMEM((CH, C), table.dtype),
            pltpu.VMEM((CH, C), table.dtype),
            pltpu.SemaphoreType.DMA,
            pltpu.SemaphoreType.DMA,
            pltpu.SemaphoreType.DMA,
            pltpu.SemaphoreType.DMA,
        ])
    def k(table_hbm, idx_hbm, out_hbm, i0, i1, r0, r1, g0, g1, w0, w1):
        wid = lax.axis_index("s") * NC + lax.axis_index("c")
        row0 = wid * rpw
        idx_v = (i0, i1)
        rows_v = (r0, r1)
        gsem = (g0, g1)
        wsem = (w0, w1)

        def pair(j, carry):
            hs = []
            for b in range(2):
                base = pl.multiple_of(row0 + (2 * j + b) * CH, 8)

                @pl.when(j > 0)
                def _():
                    pltpu.make_async_copy(
                        out_hbm.at[pl.ds(0, CH)], rows_v[b], wsem[b]).wait()

                pltpu.sync_copy(idx_hbm.at[pl.ds(base, CH)], idx_v[b])
                hs.append(pltpu.async_copy(table_hbm.at[idx_v[b]], rows_v[b],
                                           gsem[b]))
            for b in range(2):
                base = pl.multiple_of(row0 + (2 * j + b) * CH, 8)
                hs[b].wait()
                pltpu.async_copy(rows_v[b], out_hbm.at[pl.ds(base, CH)],
                                 wsem[b])
            return carry

        lax.fori_loop(0, npairs, pair, 0)
        for b in range(2):
            pltpu.make_async_copy(
                out_hbm.at[pl.ds(0, CH)], rows_v[b], wsem[b]).wait()
        if tail:
            base = pl.multiple_of(row0 + 2 * npairs * CH, 8)
            pltpu.sync_copy(idx_hbm.at[pl.ds(base, CH)], i0)
            pltpu.async_copy(table_hbm.at[i0], r0, g0).wait()
            pltpu.sync_copy(r0, out_hbm.at[pl.ds(base, CH)])

    return k(table, idx)


def _sc_scatter(vals, idx, CH):
    R = vals.shape[0]
    rpt = R // (NC * NS)
    nch = rpt // CH
    nz = NP // NS

    @functools.partial(
        pl.kernel, mesh=_mesh(),
        out_type=jax.ShapeDtypeStruct((NC, NP, D), f32),
        scratch_types=[
            pltpu.VMEM((CH,), jnp.int32),
            pltpu.VMEM((CH, D), f32),
            pltpu.VMEM_SHARED((NP, D), f32),
            pltpu.SemaphoreType.DMA,
        ])
    def k(vals_hbm, idx_hbm, zero_hbm, out_hbm, idx_v, vals_v, acc, sem):
        c = lax.axis_index("c")
        s = lax.axis_index("s")
        pltpu.sync_copy(zero_hbm.at[pl.ds(s * nz, nz)],
                        acc.at[pl.ds(s * nz, nz)])
        plsc.subcore_barrier()
        row0 = c * (R // 2) + s * rpt

        def body(i, carry):
            base = pl.multiple_of(row0 + i * CH, 8)
            pltpu.sync_copy(idx_hbm.at[pl.ds(base, CH)], idx_v)
            pltpu.sync_copy(vals_hbm.at[pl.ds(base, CH)], vals_v)
            pltpu.sync_copy(vals_v, acc.at[idx_v], add=True)
            return carry

        lax.fori_loop(0, nch, body, 0)
        plsc.subcore_barrier()
        pltpu.sync_copy(acc.at[pl.ds(s * nz, nz)],
                        out_hbm.at[c, pl.ds(s * nz, nz)])

    return k(vals, idx, jnp.zeros((NP, D), f32))[:, :N, :]


def _silu(x):
    return x * jax.nn.sigmoid(x)


def _dot(a, b):
    return jnp.dot(a, b, preferred_element_type=f32)


def _bdot(a, b):
    return jnp.dot(a.astype(jnp.bfloat16), b.astype(jnp.bfloat16),
                   preferred_element_type=f32)


def _pack2(a, b):
    ai = lax.bitcast_convert_type(a, jnp.int32)
    bi = lax.bitcast_convert_type(b, jnp.int32)
    ar = (ai + 0x7FFF + ((ai >> 16) & 1)) & (-65536)
    br = ((bi + 0x7FFF + ((bi >> 16) & 1)) >> 16) & 0xFFFF
    return ar | br


def _hi(w):
    return lax.bitcast_convert_type(w & (-65536), f32)


def _lo(w):
    return lax.bitcast_convert_type(w << 16, f32)


def _rowln(x, g, b):
    m = jnp.mean(x, axis=-1, keepdims=True)
    v = jnp.mean((x - m) * (x - m), axis=-1, keepdims=True)
    return g * (x - m) / jnp.sqrt(v + 1e-5) + b


def _gn(xc, oh, w, b, ms):
    dn = (((0,), (0,)), ((), ()))
    cnt = jnp.sum(oh, axis=0)[:, None] + 1e-6
    mean = lax.dot_general(oh, xc, dn, preferred_element_type=f32) / cnt
    out = xc - _dot(oh, mean) * ms
    var = lax.dot_general(oh, out * out, dn, preferred_element_type=f32) / cnt
    out = out / jnp.sqrt(_dot(oh, var) + 1e-5)
    return out * w + b


def _onehot(b2d):
    return (b2d == lax.broadcasted_iota(jnp.int32, (N, NG), 1)).astype(f32)


_FULL = lambda shape: pl.BlockSpec(shape, lambda *_: tuple(0 for _ in shape))


BE = 2000
GE = E // BE


def _n1_body(x_r, wd_r, ws_r, t1_r):
    x = x_r[...]
    td = _dot(x, wd_r[...])
    ts = _dot(x, ws_r[...])
    t1_r[...] = jnp.stack([_pack2(td[:, :D], td[:, D:]),
                           _pack2(ts[:, :D], ts[:, D:])], axis=0)


def _edge1_body(gd_r, gs_r, ea_r, w1e_r, b1_r, wg2_r, bg2_r, wa2_r, ba2_r,
                g_o, m_o, st_o):
    gd = gd_r[...]
    gs = gs_r[...]
    ew = _bdot(ea_r[...], w1e_r[...]) + b1_r[...]
    pre_g = _hi(gd) + _hi(gs) + ew[:, :D]
    pre_a = _lo(gd) + _lo(gs) + ew[:, D:]
    g = _bdot(_silu(pre_g), wg2_r[...]) + bg2_r[...]
    m = _bdot(_silu(pre_a), wa2_r[...]) + ba2_r[...]
    g_o[...] = g.astype(jnp.bfloat16)
    m_o[...] = m.astype(jnp.bfloat16)
    su = jnp.concatenate(
        [jnp.sum(g, 0, keepdims=True), jnp.sum(g * g, 0, keepdims=True),
         jnp.zeros((6, D), f32)], axis=0)

    @pl.when(pl.program_id(0) == 0)
    def _():
        st_o[...] = su

    @pl.when(pl.program_id(0) > 0)
    def _():
        st_o[...] = st_o[...] + su


def _edge2_body(g_r, m_r, st_r, bg_r, bb_r, sm_o):
    st = st_r[...]
    mean = st[0:1, :] / E
    var = st[1:2, :] / E - mean * mean
    sig = jax.nn.sigmoid(
        bg_r[...] * (g_r[...].astype(f32) - mean) / jnp.sqrt(var + 1e-5)
        + bb_r[...])
    sm_o[...] = jnp.stack([sig * m_r[...].astype(f32), sig], axis=0)


def _n2_body(x_r, num_r, den_r, b_r, bnxg_r, bnxb_r, gw_r, gb_r, gms_r,
             wq_r, bq_r, wk_r, bk_r, wv_r, bv_r, wsk_r, bsk_r,
             h_o, t2_o, xr_o):
    x = x_r[...]
    agg = num_r[...] / (den_r[...] + 1e-6)
    mu = jnp.mean(agg, 0, keepdims=True)
    va = jnp.mean((agg - mu) * (agg - mu), 0, keepdims=True)
    bn = bnxg_r[...] * (agg - mu) / jnp.sqrt(va + 1e-5) + bnxb_r[...]
    xcart = x + _silu(bn)
    oh = _onehot(b_r[...])
    h = x + _gn(xcart, oh, gw_r[...], gb_r[...], gms_r[...])
    h_o[...] = h
    q = _dot(h, wq_r[...]) + bq_r[...]
    kk = _dot(h, wk_r[...]) + bk_r[...]
    v = _dot(h, wv_r[...]) + bv_r[...]
    t2_o[...] = jnp.concatenate(
        [_pack2(q, kk), lax.bitcast_convert_type(v, jnp.int32)], axis=1)
    xr_o[...] = _dot(h, wsk_r[...]) + bsk_r[...]


def _edge3_body(gd_r, gs_r, ea_r, we_r, wmu1_r, wmu2_r, wmu3_r, bmu_r,
                lag_r, lab_r, wml_r, bml_r, lmg_r, lmb_r, msg_o):
    gd = gd_r[...]
    gs = gs_r[...]
    q_i = _hi(gd[:, :D])
    k_i = _lo(gd[:, :D])
    v_i = lax.bitcast_convert_type(gd[:, D:], f32)
    k_j = _lo(gs[:, :D])
    v_j = lax.bitcast_convert_type(gs[:, D:], f32)
    e = _bdot(ea_r[...], we_r[...])
    alpha = jnp.concatenate([q_i * k_i, q_i * k_j, q_i * e], axis=1)
    alpha = alpha * np.float32(1.0 / np.sqrt(3.0 * D))
    gate = jax.nn.sigmoid(_rowln(alpha, lag_r[...], lab_r[...]))
    msg0 = (_bdot(v_i, wmu1_r[...]) + _bdot(v_j, wmu2_r[...]) +
            _bdot(e, wmu3_r[...]) + bmu_r[...])
    msg1 = _bdot(msg0 * gate, wml_r[...]) + bml_r[...]
    msg_o[...] = _rowln(msg1, lmg_r[...], lmb_r[...])


def _n4_body(h_r, p_r, xr_r, b_r, wbo_r, wbx_r, gw_r, gb_r, gms_r, out_o):
    h = h_r[...]
    out = p_r[0] + p_r[1]
    xr = xr_r[...]
    beta = jax.nn.sigmoid(_dot(out, wbo_r[...]) + _dot(xr, wbx_r[...]))
    h_mat = beta * xr + (1.0 - beta) * out
    oh = _onehot(b_r[...])
    out_o[...] = h + _gn(h_mat, oh, gw_r[...], gb_r[...], gms_r[...])


def kernel(x, edge_index, edge_attr, batch, params):
    p = params
    src = edge_index[0]
    dst = edge_index[1]
    b2d = batch.reshape(N, 1)

    row2 = lambda a: a.reshape(1, -1)

    wd1 = jnp.concatenate([p['cart_Wg1'][:D], p['cart_Wa1'][:D]], axis=1)
    ws1 = jnp.concatenate([p['cart_Wg1'][D:2 * D], p['cart_Wa1'][D:2 * D]],
                          axis=1)
    w1e = jnp.concatenate([p['cart_Wg1'][2 * D:], p['cart_Wa1'][2 * D:]],
                          axis=1)
    b1 = row2(jnp.concatenate([p['cart_bg1'], p['cart_ba1']]))
    wbo = p['mat_Wbeta'][:D] + p['mat_Wbeta'][2 * D:]
    wbx = p['mat_Wbeta'][D:2 * D] - p['mat_Wbeta'][2 * D:]

    t1 = pl.pallas_call(
        _n1_body,
        grid=(),
        in_specs=[_FULL((N, D)), _FULL((D, 2 * D)), _FULL((D, 2 * D))],
        out_specs=_FULL((2, N, D)),
        out_shape=jax.ShapeDtypeStruct((2, N, D), jnp.int32),
    )(x, wd1, ws1)

    idx1 = jnp.concatenate([dst, src + N])
    g1 = _sc_gather(t1.reshape(2 * N, D), idx1, D, 400)

    espec = lambda w: pl.BlockSpec((BE, w), lambda i: (i, 0))
    espec_hi = lambda w: pl.BlockSpec((BE, w), lambda i: (GE + i, 0))
    g_arr, m_arr, st = pl.pallas_call(
        _edge1_body,
        grid=(GE,),
        in_specs=[espec(D), espec_hi(D), espec(D),
                  _FULL((D, 2 * D)), _FULL((1, 2 * D)),
                  _FULL((D, D)), _FULL((1, D)),
                  _FULL((D, D)), _FULL((1, D))],
        out_specs=[espec(D), espec(D),
                   pl.BlockSpec((8, D), lambda i: (0, 0))],
        out_shape=[jax.ShapeDtypeStruct((E, D), jnp.bfloat16),
                   jax.ShapeDtypeStruct((E, D), jnp.bfloat16),
                   jax.ShapeDtypeStruct((8, D), f32)],
    )(g1, g1, edge_attr, w1e, b1,
      p['cart_Wg2'], row2(p['cart_bg2']), p['cart_Wa2'], row2(p['cart_ba2']))

    sm = pl.pallas_call(
        _edge2_body,
        grid=(GE,),
        in_specs=[espec(D), espec(D), _FULL((8, D)),
                  _FULL((1, D)), _FULL((1, D))],
        out_specs=pl.BlockSpec((2, BE, D), lambda i: (0, i, 0)),
        out_shape=jax.ShapeDtypeStruct((2, E, D), f32),
    )(g_arr, m_arr, st, row2(p['cart_bne_g']), row2(p['cart_bne_b']))

    nd = _sc_scatter(sm.reshape(2 * E, D), jnp.concatenate([dst, dst]), 200)
    num = nd[0]
    den = nd[1]

    h, t2, xr = pl.pallas_call(
        _n2_body,
        grid=(),
        in_specs=[_FULL((N, D)), _FULL((N, D)), _FULL((N, D)),
                  _FULL((N, 1)),
                  _FULL((1, D)), _FULL((1, D)),
                  _FULL((1, D)), _FULL((1, D)), _FULL((1, D)),
                  _FULL((D, D)), _FULL((1, D)),
                  _FULL((D, D)), _FULL((1, D)),
                  _FULL((D, D)), _FULL((1, D)),
                  _FULL((D, D)), _FULL((1, D))],
        out_specs=[_FULL((N, D)), _FULL((N, 2 * D)), _FULL((N, D))],
        out_shape=[jax.ShapeDtypeStruct((N, D), f32),
                   jax.ShapeDtypeStruct((N, 2 * D), jnp.int32),
                   jax.ShapeDtypeStruct((N, D), f32)],
    )(x, num, den, b2d,
      row2(p['cart_bnx_g']), row2(p['cart_bnx_b']),
      row2(p['gnc_w']), row2(p['gnc_b']), row2(p['gnc_ms']),
      p['mat_Wq'], row2(p['mat_bq']), p['mat_Wk'], row2(p['mat_bk']),
      p['mat_Wv'], row2(p['mat_bv']), p['mat_Wskip'], row2(p['mat_bskip']))

    idx2 = jnp.concatenate([dst, src])
    g2 = _sc_gather(t2, idx2, 2 * D, 200)

    msg = pl.pallas_call(
        _edge3_body,
        grid=(GE,),
        in_specs=[espec(2 * D), espec_hi(2 * D), espec(D),
                  _FULL((D, D)),
                  _FULL((D, 3 * D)), _FULL((D, 3 * D)), _FULL((D, 3 * D)),
                  _FULL((1, 3 * D)), _FULL((1, 3 * D)), _FULL((1, 3 * D)),
                  _FULL((3 * D, D)), _FULL((1, D)),
                  _FULL((1, D)), _FULL((1, D))],
        out_specs=espec(D),
        out_shape=jax.ShapeDtypeStruct((E, D), f32),
    )(g2, g2, edge_attr, p['mat_We'],
      p['mat_Wmu'][:D], p['mat_Wmu'][D:2 * D], p['mat_Wmu'][2 * D:],
      row2(p['mat_bmu']), row2(p['mat_lna_g']), row2(p['mat_lna_b']),
      p['mat_Wml'], row2(p['mat_bml']),
      row2(p['mat_lnm_g']), row2(p['mat_lnm_b']))

    mo = _sc_scatter(msg, dst, 200)

    return pl.pallas_call(
        _n4_body,
        grid=(),
        in_specs=[_FULL((N, D)), _FULL((2, N, D)), _FULL((N, D)),
                  _FULL((N, 1)), _FULL((D, 1)), _FULL((D, 1)),
                  _FULL((1, D)), _FULL((1, D)), _FULL((1, D))],
        out_specs=_FULL((N, D)),
        out_shape=jax.ShapeDtypeStruct((N, D), f32),
    )(h, mo, xr, b2d, wbo, wbx,
      row2(p['gnm_w']), row2(p['gnm_b']), row2(p['gnm_ms']))


# speedup vs baseline: 3.9414x; 1.0124x over previous
import functools

import jax
import jax.numpy as jnp
import numpy as np
from jax import lax
from jax.experimental import pallas as pl
from jax.experimental.pallas import tpu as pltpu
from jax.experimental.pallas import tpu_sc as plsc

N = 10000
E = 160000
D = 128
NG = 64
NC = 2
NS = 16
f32 = jnp.float32

_mesh = functools.partial(
    plsc.VectorSubcoreMesh, core_axis_name="c", subcore_axis_name="s")


NP = 10240


def _sc_gather(table, idx, C, CH):
    R = idx.shape[0]
    rpw = R // (NC * NS)
    nch = rpw // CH
    npairs = nch // 2
    tail = nch % 2 == 1

    @functools.partial(
        pl.kernel, mesh=_mesh(),
        out_type=jax.ShapeDtypeStruct((R, C), table.dtype),
        scratch_types=[
            pltpu.VMEM((CH,), jnp.int32),
            pltpu.VMEM((CH,), jnp.int32),
            pltpu.VMEM((CH, C), table.dtype),
            pltpu.VMEM((CH, C), table.dtype),
            pltpu.SemaphoreType.DMA,
            pltpu.SemaphoreType.DMA,
            pltpu.SemaphoreType.DMA,
            pltpu.SemaphoreType.DMA,
        ])
    def k(table_hbm, idx_hbm, out_hbm, i0, i1, r0, r1, g0, g1, w0, w1):
        wid = lax.axis_index("s") * NC + lax.axis_index("c")
        row0 = wid * rpw
        idx_v = (i0, i1)
        rows_v = (r0, r1)
        gsem = (g0, g1)
        wsem = (w0, w1)

        def pair(j, carry):
            hs = []
            for b in range(2):
                base = pl.multiple_of(row0 + (2 * j + b) * CH, 8)

                @pl.when(j > 0)
                def _():
                    pltpu.make_async_copy(
                        out_hbm.at[pl.ds(0, CH)], rows_v[b], wsem[b]).wait()

                pltpu.sync_copy(idx_hbm.at[pl.ds(base, CH)], idx_v[b])
                hs.append(pltpu.async_copy(table_hbm.at[idx_v[b]], rows_v[b],
                                           gsem[b]))
            for b in range(2):
                base = pl.multiple_of(row0 + (2 * j + b) * CH, 8)
                hs[b].wait()
                pltpu.async_copy(rows_v[b], out_hbm.at[pl.ds(base, CH)],
                                 wsem[b])
            return carry

        lax.fori_loop(0, npairs, pair, 0)
        for b in range(2):
            pltpu.make_async_copy(
                out_hbm.at[pl.ds(0, CH)], rows_v[b], wsem[b]).wait()
        if tail:
            base = pl.multiple_of(row0 + 2 * npairs * CH, 8)
            pltpu.sync_copy(idx_hbm.at[pl.ds(base, CH)], i0)
            pltpu.async_copy(table_hbm.at[i0], r0, g0).wait()
            pltpu.sync_copy(r0, out_hbm.at[pl.ds(base, CH)])

    return k(table, idx)


def _sc_scatter(vals, idx, CH):
    R = vals.shape[0]
    rpt = R // (NC * NS)
    nch = rpt // CH
    nz = NP // NS

    @functools.partial(
        pl.kernel, mesh=_mesh(),
        out_type=jax.ShapeDtypeStruct((NC, NP, D), f32),
        scratch_types=[
            pltpu.VMEM((CH,), jnp.int32),
            pltpu.VMEM((CH, D), f32),
            pltpu.VMEM_SHARED((NP, D), f32),
            pltpu.SemaphoreType.DMA,
        ])
    def k(vals_hbm, idx_hbm, zero_hbm, out_hbm, idx_v, vals_v, acc, sem):
        c = lax.axis_index("c")
        s = lax.axis_index("s")
        pltpu.sync_copy(zero_hbm.at[pl.ds(s * nz, nz)],
                        acc.at[pl.ds(s * nz, nz)])
        plsc.subcore_barrier()
        row0 = c * (R // 2) + s * rpt

        def body(i, carry):
            base = pl.multiple_of(row0 + i * CH, 8)
            pltpu.sync_copy(idx_hbm.at[pl.ds(base, CH)], idx_v)
            pltpu.sync_copy(vals_hbm.at[pl.ds(base, CH)], vals_v)
            pltpu.sync_copy(vals_v, acc.at[idx_v], add=True)
            return carry

        lax.fori_loop(0, nch, body, 0)
        plsc.subcore_barrier()
        pltpu.sync_copy(acc.at[pl.ds(s * nz, nz)],
                        out_hbm.at[c, pl.ds(s * nz, nz)])

    return k(vals, idx, jnp.zeros((NP, D), f32))[:, :N, :]


def _silu(x):
    return x * jax.nn.sigmoid(x)


def _dot(a, b):
    return jnp.dot(a, b, preferred_element_type=f32)


def _bdot(a, b):
    return jnp.dot(a.astype(jnp.bfloat16), b.astype(jnp.bfloat16),
                   preferred_element_type=f32)


def _pack2(a, b):
    ai = lax.bitcast_convert_type(a, jnp.int32)
    bi = lax.bitcast_convert_type(b, jnp.int32)
    ar = (ai + 0x7FFF + ((ai >> 16) & 1)) & (-65536)
    br = ((bi + 0x7FFF + ((bi >> 16) & 1)) >> 16) & 0xFFFF
    return ar | br


def _hi(w):
    return lax.bitcast_convert_type(w & (-65536), f32)


def _lo(w):
    return lax.bitcast_convert_type(w << 16, f32)


def _rowln(x, g, b):
    m = jnp.mean(x, axis=-1, keepdims=True)
    v = jnp.mean((x - m) * (x - m), axis=-1, keepdims=True)
    return g * (x - m) / jnp.sqrt(v + 1e-5) + b


def _gn(xc, oh, w, b, ms):
    dn = (((0,), (0,)), ((), ()))
    cnt = jnp.sum(oh, axis=0)[:, None] + 1e-6
    mean = lax.dot_general(oh, xc, dn, preferred_element_type=f32) / cnt
    out = xc - _dot(oh, mean) * ms
    var = lax.dot_general(oh, out * out, dn, preferred_element_type=f32) / cnt
    out = out / jnp.sqrt(_dot(oh, var) + 1e-5)
    return out * w + b


def _onehot(b2d):
    return (b2d == lax.broadcasted_iota(jnp.int32, (N, NG), 1)).astype(f32)


_FULL = lambda shape: pl.BlockSpec(shape, lambda *_: tuple(0 for _ in shape))


BE = 2000
GE = E // BE


def _n1_body(x_r, wd_r, ws_r, t1_r):
    x = x_r[...]
    td = _dot(x, wd_r[...])
    ts = _dot(x, ws_r[...])
    t1_r[...] = jnp.stack([_pack2(td[:, :D], td[:, D:]),
                           _pack2(ts[:, :D], ts[:, D:])], axis=0)


def _edge1_body(gd_r, gs_r, ea_r, w1e_r, b1_r, wg2_r, bg2_r, wa2_r, ba2_r,
                g_o, m_o, st_o):
    gd = gd_r[...]
    gs = gs_r[...]
    ew = _bdot(ea_r[...], w1e_r[...]) + b1_r[...]
    pre_g = _hi(gd) + _hi(gs) + ew[:, :D]
    pre_a = _lo(gd) + _lo(gs) + ew[:, D:]
    g = _bdot(_silu(pre_g), wg2_r[...]) + bg2_r[...]
    m = _bdot(_silu(pre_a), wa2_r[...]) + ba2_r[...]
    g_o[...] = g.astype(jnp.bfloat16)
    m_o[...] = m.astype(jnp.bfloat16)
    su = jnp.concatenate(
        [jnp.sum(g, 0, keepdims=True), jnp.sum(g * g, 0, keepdims=True),
         jnp.zeros((6, D), f32)], axis=0)

    @pl.when(pl.program_id(0) == 0)
    def _():
        st_o[...] = su

    @pl.when(pl.program_id(0) > 0)
    def _():
        st_o[...] = st_o[...] + su


def _edge2_body(g_r, m_r, sta_r, stb_r, bg_r, bb_r, sm_o):
    st = sta_r[...] + stb_r[...]
    mean = st[0:1, :] / E
    var = st[1:2, :] / E - mean * mean
    sig = jax.nn.sigmoid(
        bg_r[...] * (g_r[...].astype(f32) - mean) / jnp.sqrt(var + 1e-5)
        + bb_r[...])
    sm_o[...] = jnp.stack([sig * m_r[...].astype(f32), sig], axis=0)


def _n2_body(x_r, nd_r, b_r, bnxg_r, bnxb_r, gw_r, gb_r, gms_r,
             wq_r, bq_r, wk_r, bk_r, wv_r, bv_r, wsk_r, bsk_r,
             h_o, t2_o, xr_o):
    x = x_r[...]
    agg = nd_r[0] / (nd_r[1] + 1e-6)
    mu = jnp.mean(agg, 0, keepdims=True)
    va = jnp.mean((agg - mu) * (agg - mu), 0, keepdims=True)
    bn = bnxg_r[...] * (agg - mu) / jnp.sqrt(va + 1e-5) + bnxb_r[...]
    xcart = x + _silu(bn)
    oh = _onehot(b_r[...])
    h = x + _gn(xcart, oh, gw_r[...], gb_r[...], gms_r[...])
    h_o[...] = h
    q = _dot(h, wq_r[...]) + bq_r[...]
    kk = _dot(h, wk_r[...]) + bk_r[...]
    v = _dot(h, wv_r[...]) + bv_r[...]
    t2_o[...] = jnp.concatenate(
        [_pack2(q, kk), lax.bitcast_convert_type(v, jnp.int32)], axis=1)
    xr_o[...] = _dot(h, wsk_r[...]) + bsk_r[...]


def _edge3_body(gd_r, gs_r, ea_r, we_r, wmu1_r, wmu2_r, wmu3_r, bmu_r,
                lag_r, lab_r, wml_r, bml_r, lmg_r, lmb_r, msg_o):
    gd = gd_r[...]
    gs = gs_r[...]
    q_i = _hi(gd[:, :D])
    k_i = _lo(gd[:, :D])
    v_i = lax.bitcast_convert_type(gd[:, D:], f32)
    k_j = _lo(gs[:, :D])
    v_j = lax.bitcast_convert_type(gs[:, D:], f32)
    e = _bdot(ea_r[...], we_r[...])
    alpha = jnp.concatenate([q_i * k_i, q_i * k_j, q_i * e], axis=1)
    alpha = alpha * np.float32(1.0 / np.sqrt(3.0 * D))
    gate = jax.nn.sigmoid(_rowln(alpha, lag_r[...], lab_r[...]))
    msg0 = (_bdot(v_i, wmu1_r[...]) + _bdot(v_j, wmu2_r[...]) +
            _bdot(e, wmu3_r[...]) + bmu_r[...])
    msg1 = _bdot(msg0 * gate, wml_r[...]) + bml_r[...]
    msg_o[...] = _rowln(msg1, lmg_r[...], lmb_r[...])


def _n4_body(h_r, p_r, xr_r, b_r, wbo_r, wbx_r, gw_r, gb_r, gms_r, out_o):
    h = h_r[...]
    out = p_r[0] + p_r[1]
    xr = xr_r[...]
    beta = jax.nn.sigmoid(_dot(out, wbo_r[...]) + _dot(xr, wbx_r[...]))
    h_mat = beta * xr + (1.0 - beta) * out
    oh = _onehot(b_r[...])
    out_o[...] = h + _gn(h_mat, oh, gw_r[...], gb_r[...], gms_r[...])


def kernel(x, edge_index, edge_attr, batch, params):
    p = params
    src = edge_index[0]
    dst = edge_index[1]
    b2d = batch.reshape(N, 1)

    row2 = lambda a: a.reshape(1, -1)

    wd1 = jnp.concatenate([p['cart_Wg1'][:D], p['cart_Wa1'][:D]], axis=1)
    ws1 = jnp.concatenate([p['cart_Wg1'][D:2 * D], p['cart_Wa1'][D:2 * D]],
                          axis=1)
    w1e = jnp.concatenate([p['cart_Wg1'][2 * D:], p['cart_Wa1'][2 * D:]],
                          axis=1)
    b1 = row2(jnp.concatenate([p['cart_bg1'], p['cart_ba1']]))
    wbo = p['mat_Wbeta'][:D] + p['mat_Wbeta'][2 * D:]
    wbx = p['mat_Wbeta'][D:2 * D] - p['mat_Wbeta'][2 * D:]

    t1 = pl.pallas_call(
        _n1_body,
        grid=(),
        in_specs=[_FULL((N, D)), _FULL((D, 2 * D)), _FULL((D, 2 * D))],
        out_specs=_FULL((2, N, D)),
        out_shape=jax.ShapeDtypeStruct((2, N, D), jnp.int32),
    )(x, wd1, ws1)

    EH = E // 2
    GEh = EH // BE
    ea_b = edge_attr.astype(jnp.bfloat16)
    espec = lambda w: pl.BlockSpec((BE, w), lambda i: (i, 0))
    espec_at = lambda w, off: pl.BlockSpec((BE, w), lambda i: (off + i, 0))
    t1f = t1.reshape(2 * N, D)

    def cart_half(h):
        lo = h * EH
        idx1 = jnp.concatenate([lax.dynamic_slice(dst, (lo,), (EH,)),
                                lax.dynamic_slice(src, (lo,), (EH,)) + N])
        g1 = _sc_gather(t1f, idx1, D, 200)
        return pl.pallas_call(
            _edge1_body,
            grid=(GEh,),
            in_specs=[espec(D), espec_at(D, GEh), espec_at(D, h * GEh),
                      _FULL((D, 2 * D)), _FULL((1, 2 * D)),
                      _FULL((D, D)), _FULL((1, D)),
                      _FULL((D, D)), _FULL((1, D))],
            out_specs=[espec(D), espec(D),
                       pl.BlockSpec((8, D), lambda i: (0, 0))],
            out_shape=[jax.ShapeDtypeStruct((EH, D), jnp.bfloat16),
                       jax.ShapeDtypeStruct((EH, D), jnp.bfloat16),
                       jax.ShapeDtypeStruct((8, D), f32)],
        )(g1, g1, ea_b, w1e, b1,
          p['cart_Wg2'], row2(p['cart_bg2']),
          p['cart_Wa2'], row2(p['cart_ba2']))

    ga, ma, sta = cart_half(0)
    gb, mb, stb = cart_half(1)

    def cart_tail(h, g_arr, m_arr):
        lo = h * EH
        sm = pl.pallas_call(
            _edge2_body,
            grid=(GEh,),
            in_specs=[espec(D), espec(D), _FULL((8, D)), _FULL((8, D)),
                      _FULL((1, D)), _FULL((1, D))],
            out_specs=pl.BlockSpec((2, BE, D), lambda i: (0, i, 0)),
            out_shape=jax.ShapeDtypeStruct((2, EH, D), f32),
        )(g_arr, m_arr, sta, stb,
          row2(p['cart_bne_g']), row2(p['cart_bne_b']))
        dh = lax.dynamic_slice(dst, (lo,), (EH,))
        return _sc_scatter(sm.reshape(2 * EH, D),
                           jnp.concatenate([dh, dh]), 200)

    nd_a = cart_tail(0, ga, ma)
    nd_b = cart_tail(1, gb, mb)

    h, t2, xr = pl.pallas_call(
        _n2_body,
        grid=(),
        in_specs=[_FULL((N, D)), _FULL((2, N, D)),
                  _FULL((N, 1)),
                  _FULL((1, D)), _FULL((1, D)),
                  _FULL((1, D)), _FULL((1, D)), _FULL((1, D)),
                  _FULL((D, D)), _FULL((1, D)),
                  _FULL((D, D)), _FULL((1, D)),
                  _FULL((D, D)), _FULL((1, D)),
                  _FULL((D, D)), _FULL((1, D))],
        out_specs=[_FULL((N, D)), _FULL((N, 2 * D)), _FULL((N, D))],
        out_shape=[jax.ShapeDtypeStruct((N, D), f32),
                   jax.ShapeDtypeStruct((N, 2 * D), jnp.int32),
                   jax.ShapeDtypeStruct((N, D), f32)],
    )(x, nd_a + nd_b, b2d,
      row2(p['cart_bnx_g']), row2(p['cart_bnx_b']),
      row2(p['gnc_w']), row2(p['gnc_b']), row2(p['gnc_ms']),
      p['mat_Wq'], row2(p['mat_bq']), p['mat_Wk'], row2(p['mat_bk']),
      p['mat_Wv'], row2(p['mat_bv']), p['mat_Wskip'], row2(p['mat_bskip']))

    def mat_half(h):
        lo = h * EH
        idx2 = jnp.concatenate([lax.dynamic_slice(dst, (lo,), (EH,)),
                                lax.dynamic_slice(src, (lo,), (EH,))])
        g2 = _sc_gather(t2, idx2, 2 * D, 200)
        return pl.pallas_call(
            _edge3_body,
            grid=(GEh,),
            in_specs=[espec(2 * D), espec_at(2 * D, GEh), espec_at(D, h * GEh),
                      _FULL((D, D)),
                      _FULL((D, 3 * D)), _FULL((D, 3 * D)), _FULL((D, 3 * D)),
                      _FULL((1, 3 * D)), _FULL((1, 3 * D)), _FULL((1, 3 * D)),
                      _FULL((3 * D, D)), _FULL((1, D)),
                      _FULL((1, D)), _FULL((1, D))],
            out_specs=espec(D),
            out_shape=jax.ShapeDtypeStruct((EH, D), f32),
        )(g2, g2, ea_b, p['mat_We'],
          p['mat_Wmu'][:D], p['mat_Wmu'][D:2 * D], p['mat_Wmu'][2 * D:],
          row2(p['mat_bmu']), row2(p['mat_lna_g']), row2(p['mat_lna_b']),
          p['mat_Wml'], row2(p['mat_bml']),
          row2(p['mat_lnm_g']), row2(p['mat_lnm_b']))

    msg = jnp.concatenate([mat_half(0), mat_half(1)], axis=0)

    mo = _sc_scatter(msg, dst, 200)

    return pl.pallas_call(
        _n4_body,
        grid=(),
        in_specs=[_FULL((N, D)), _FULL((2, N, D)), _FULL((N, D)),
                  _FULL((N, 1)), _FULL((D, 1)), _FULL((D, 1)),
                  _FULL((1, D)), _FULL((1, D)), _FULL((1, D))],
        out_specs=_FULL((N, D)),
        out_shape=jax.ShapeDtypeStruct((N, D), f32),
    )(h, mo, xr, b2d, wbo, wbx,
      row2(p['gnm_w']), row2(p['gnm_b']), row2(p['gnm_ms']))


# speedup vs baseline: 4.1025x; 1.0409x over previous
import functools

import jax
import jax.numpy as jnp
import numpy as np
from jax import lax
from jax.experimental import pallas as pl
from jax.experimental.pallas import tpu as pltpu
from jax.experimental.pallas import tpu_sc as plsc

N = 10000
E = 160000
D = 128
NG = 64
NC = 2
NS = 16
f32 = jnp.float32

_mesh = functools.partial(
    plsc.VectorSubcoreMesh, core_axis_name="c", subcore_axis_name="s")


NP = 10240


def _sc_gather(table, idx, C, CH):
    R = idx.shape[0]
    rpw = R // (NC * NS)
    nch = rpw // CH
    npairs = nch // 2
    tail = nch % 2 == 1

    @functools.partial(
        pl.kernel, mesh=_mesh(),
        out_type=jax.ShapeDtypeStruct((R, C), table.dtype),
        scratch_types=[
            pltpu.VMEM((CH,), jnp.int32),
            pltpu.VMEM((CH,), jnp.int32),
            pltpu.VMEM((CH, C), table.dtype),
            pltpu.VMEM((CH, C), table.dtype),
            pltpu.SemaphoreType.DMA,
            pltpu.SemaphoreType.DMA,
            pltpu.SemaphoreType.DMA,
            pltpu.SemaphoreType.DMA,
        ])
    def k(table_hbm, idx_hbm, out_hbm, i0, i1, r0, r1, g0, g1, w0, w1):
        wid = lax.axis_index("s") * NC + lax.axis_index("c")
        row0 = wid * rpw
        idx_v = (i0, i1)
        rows_v = (r0, r1)
        gsem = (g0, g1)
        wsem = (w0, w1)

        def pair(j, carry):
            hs = []
            for b in range(2):
                base = pl.multiple_of(row0 + (2 * j + b) * CH, 8)

                @pl.when(j > 0)
                def _():
                    pltpu.make_async_copy(
                        out_hbm.at[pl.ds(0, CH)], rows_v[b], wsem[b]).wait()

                pltpu.sync_copy(idx_hbm.at[pl.ds(base, CH)], idx_v[b])
                hs.append(pltpu.async_copy(table_hbm.at[idx_v[b]], rows_v[b],
                                           gsem[b]))
            for b in range(2):
                base = pl.multiple_of(row0 + (2 * j + b) * CH, 8)
                hs[b].wait()
                pltpu.async_copy(rows_v[b], out_hbm.at[pl.ds(base, CH)],
                                 wsem[b])
            return carry

        lax.fori_loop(0, npairs, pair, 0)
        for b in range(2):
            pltpu.make_async_copy(
                out_hbm.at[pl.ds(0, CH)], rows_v[b], wsem[b]).wait()
        if tail:
            base = pl.multiple_of(row0 + 2 * npairs * CH, 8)
            pltpu.sync_copy(idx_hbm.at[pl.ds(base, CH)], i0)
            pltpu.async_copy(table_hbm.at[i0], r0, g0).wait()
            pltpu.sync_copy(r0, out_hbm.at[pl.ds(base, CH)])

    return k(table, idx)


def _sc_scatter(vals, idx, CH):
    R = vals.shape[0]
    rpt = R // (NC * NS)
    nch = rpt // CH
    nz = NP // NS

    @functools.partial(
        pl.kernel, mesh=_mesh(),
        out_type=jax.ShapeDtypeStruct((NC, NP, D), f32),
        scratch_types=[
            pltpu.VMEM((CH,), jnp.int32),
            pltpu.VMEM((CH, D), f32),
            pltpu.VMEM_SHARED((NP, D), f32),
            pltpu.SemaphoreType.DMA,
        ])
    def k(vals_hbm, idx_hbm, zero_hbm, out_hbm, idx_v, vals_v, acc, sem):
        c = lax.axis_index("c")
        s = lax.axis_index("s")
        pltpu.sync_copy(zero_hbm.at[pl.ds(s * nz, nz)],
                        acc.at[pl.ds(s * nz, nz)])
        plsc.subcore_barrier()
        row0 = c * (R // 2) + s * rpt

        def body(i, carry):
            base = pl.multiple_of(row0 + i * CH, 8)
            pltpu.sync_copy(idx_hbm.at[pl.ds(base, CH)], idx_v)
            pltpu.sync_copy(vals_hbm.at[pl.ds(base, CH)], vals_v)
            pltpu.sync_copy(vals_v, acc.at[idx_v], add=True)
            return carry

        lax.fori_loop(0, nch, body, 0)
        plsc.subcore_barrier()
        pltpu.sync_copy(acc.at[pl.ds(s * nz, nz)],
                        out_hbm.at[c, pl.ds(s * nz, nz)])

    return k(vals, idx, jnp.zeros((NP, D), f32))[:, :N, :]


def _silu(x):
    return x * jax.nn.sigmoid(x)


def _dot(a, b):
    return jnp.dot(a, b, preferred_element_type=f32)


def _bdot(a, b):
    return jnp.dot(a.astype(jnp.bfloat16), b.astype(jnp.bfloat16),
                   preferred_element_type=f32)


def _pack2(a, b):
    ai = lax.bitcast_convert_type(a, jnp.int32)
    bi = lax.bitcast_convert_type(b, jnp.int32)
    ar = (ai + 0x7FFF + ((ai >> 16) & 1)) & (-65536)
    br = ((bi + 0x7FFF + ((bi >> 16) & 1)) >> 16) & 0xFFFF
    return ar | br


def _hi(w):
    return lax.bitcast_convert_type(w & (-65536), f32)


def _lo(w):
    return lax.bitcast_convert_type(w << 16, f32)


def _rowln(x, g, b):
    m = jnp.mean(x, axis=-1, keepdims=True)
    v = jnp.mean((x - m) * (x - m), axis=-1, keepdims=True)
    return g * (x - m) / jnp.sqrt(v + 1e-5) + b


def _gn(xc, oh, w, b, ms):
    dn = (((0,), (0,)), ((), ()))
    cnt = jnp.sum(oh, axis=0)[:, None] + 1e-6
    mean = lax.dot_general(oh, xc, dn, preferred_element_type=f32) / cnt
    out = xc - _dot(oh, mean) * ms
    var = lax.dot_general(oh, out * out, dn, preferred_element_type=f32) / cnt
    out = out / jnp.sqrt(_dot(oh, var) + 1e-5)
    return out * w + b


def _onehot(b2d):
    return (b2d == lax.broadcasted_iota(jnp.int32, (N, NG), 1)).astype(f32)


_FULL = lambda shape: pl.BlockSpec(shape, lambda *_: tuple(0 for _ in shape))


BE = 4000
GE = E // BE


def _n1_body(x_r, wd_r, ws_r, t1_r):
    x = x_r[...]
    td = _dot(x, wd_r[...])
    ts = _dot(x, ws_r[...])
    t1_r[...] = jnp.stack([_pack2(td[:, :D], td[:, D:]),
                           _pack2(ts[:, :D], ts[:, D:])], axis=0)


def _edge1_body(gd_r, gs_r, ea_r, w1e_r, b1_r, wg2_r, bg2_r, wa2_r, ba2_r,
                g_o, m_o, st_o):
    gd = gd_r[...]
    gs = gs_r[...]
    ew = _bdot(ea_r[...], w1e_r[...]) + b1_r[...]
    pre_g = _hi(gd) + _hi(gs) + ew[:, :D]
    pre_a = _lo(gd) + _lo(gs) + ew[:, D:]
    g = _bdot(_silu(pre_g), wg2_r[...]) + bg2_r[...]
    m = _bdot(_silu(pre_a), wa2_r[...]) + ba2_r[...]
    g_o[...] = g.astype(jnp.bfloat16)
    m_o[...] = m.astype(jnp.bfloat16)
    su = jnp.concatenate(
        [jnp.sum(g, 0, keepdims=True), jnp.sum(g * g, 0, keepdims=True),
         jnp.zeros((6, D), f32)], axis=0)

    @pl.when(pl.program_id(0) == 0)
    def _():
        st_o[...] = su

    @pl.when(pl.program_id(0) > 0)
    def _():
        st_o[...] = st_o[...] + su


def _edge2_body(g_r, m_r, sta_r, stb_r, bg_r, bb_r, sm_o):
    st = sta_r[...] + stb_r[...]
    mean = st[0:1, :] / E
    var = st[1:2, :] / E - mean * mean
    sig = jax.nn.sigmoid(
        bg_r[...] * (g_r[...].astype(f32) - mean) / jnp.sqrt(var + 1e-5)
        + bb_r[...])
    sm_o[...] = jnp.stack([sig * m_r[...].astype(f32), sig], axis=0)


def _n2_body(x_r, nd_r, b_r, bnxg_r, bnxb_r, gw_r, gb_r, gms_r,
             wq_r, bq_r, wk_r, bk_r, wv_r, bv_r, wsk_r, bsk_r,
             h_o, t2_o, xr_o):
    x = x_r[...]
    agg = nd_r[0] / (nd_r[1] + 1e-6)
    mu = jnp.mean(agg, 0, keepdims=True)
    va = jnp.mean((agg - mu) * (agg - mu), 0, keepdims=True)
    bn = bnxg_r[...] * (agg - mu) / jnp.sqrt(va + 1e-5) + bnxb_r[...]
    xcart = x + _silu(bn)
    oh = _onehot(b_r[...])
    h = x + _gn(xcart, oh, gw_r[...], gb_r[...], gms_r[...])
    h_o[...] = h
    q = _dot(h, wq_r[...]) + bq_r[...]
    kk = _dot(h, wk_r[...]) + bk_r[...]
    v = _dot(h, wv_r[...]) + bv_r[...]
    t2_o[...] = jnp.concatenate(
        [_pack2(q, kk), lax.bitcast_convert_type(v, jnp.int32)], axis=1)
    xr_o[...] = _dot(h, wsk_r[...]) + bsk_r[...]


def _edge3_body(gd_r, gs_r, ea_r, we_r, wmu1_r, wmu2_r, wmu3_r, bmu_r,
                lag_r, lab_r, wml_r, bml_r, lmg_r, lmb_r, msg_o):
    gd = gd_r[...]
    gs = gs_r[...]
    q_i = _hi(gd[:, :D])
    k_i = _lo(gd[:, :D])
    v_i = lax.bitcast_convert_type(gd[:, D:], f32)
    k_j = _lo(gs[:, :D])
    v_j = lax.bitcast_convert_type(gs[:, D:], f32)
    e = _bdot(ea_r[...], we_r[...])
    alpha = jnp.concatenate([q_i * k_i, q_i * k_j, q_i * e], axis=1)
    alpha = alpha * np.float32(1.0 / np.sqrt(3.0 * D))
    gate = jax.nn.sigmoid(_rowln(alpha, lag_r[...], lab_r[...]))
    msg0 = (_bdot(v_i, wmu1_r[...]) + _bdot(v_j, wmu2_r[...]) +
            _bdot(e, wmu3_r[...]) + bmu_r[...])
    msg1 = _bdot(msg0 * gate, wml_r[...]) + bml_r[...]
    msg_o[...] = _rowln(msg1, lmg_r[...], lmb_r[...])


def _n4_body(h_r, p_r, xr_r, b_r, wbo_r, wbx_r, gw_r, gb_r, gms_r, out_o):
    h = h_r[...]
    out = p_r[0] + p_r[1]
    xr = xr_r[...]
    beta = jax.nn.sigmoid(_dot(out, wbo_r[...]) + _dot(xr, wbx_r[...]))
    h_mat = beta * xr + (1.0 - beta) * out
    oh = _onehot(b_r[...])
    out_o[...] = h + _gn(h_mat, oh, gw_r[...], gb_r[...], gms_r[...])


def kernel(x, edge_index, edge_attr, batch, params):
    p = params
    src = edge_index[0]
    dst = edge_index[1]
    b2d = batch.reshape(N, 1)

    row2 = lambda a: a.reshape(1, -1)

    wd1 = jnp.concatenate([p['cart_Wg1'][:D], p['cart_Wa1'][:D]], axis=1)
    ws1 = jnp.concatenate([p['cart_Wg1'][D:2 * D], p['cart_Wa1'][D:2 * D]],
                          axis=1)
    w1e = jnp.concatenate([p['cart_Wg1'][2 * D:], p['cart_Wa1'][2 * D:]],
                          axis=1)
    b1 = row2(jnp.concatenate([p['cart_bg1'], p['cart_ba1']]))
    wbo = p['mat_Wbeta'][:D] + p['mat_Wbeta'][2 * D:]
    wbx = p['mat_Wbeta'][D:2 * D] - p['mat_Wbeta'][2 * D:]

    t1 = pl.pallas_call(
        _n1_body,
        grid=(),
        in_specs=[_FULL((N, D)), _FULL((D, 2 * D)), _FULL((D, 2 * D))],
        out_specs=_FULL((2, N, D)),
        out_shape=jax.ShapeDtypeStruct((2, N, D), jnp.int32),
    )(x, wd1, ws1)

    EH = E // 2
    GEh = EH // BE
    ea_b = edge_attr.astype(jnp.bfloat16)
    espec = lambda w: pl.BlockSpec((BE, w), lambda i: (i, 0))
    espec_at = lambda w, off: pl.BlockSpec((BE, w), lambda i: (off + i, 0))
    t1f = t1.reshape(2 * N, D)

    def cart_half(h):
        lo = h * EH
        idx1 = jnp.concatenate([lax.dynamic_slice(dst, (lo,), (EH,)),
                                lax.dynamic_slice(src, (lo,), (EH,)) + N])
        g1 = _sc_gather(t1f, idx1, D, 200)
        return pl.pallas_call(
            _edge1_body,
            grid=(GEh,),
            in_specs=[espec(D), espec_at(D, GEh), espec_at(D, h * GEh),
                      _FULL((D, 2 * D)), _FULL((1, 2 * D)),
                      _FULL((D, D)), _FULL((1, D)),
                      _FULL((D, D)), _FULL((1, D))],
            out_specs=[espec(D), espec(D),
                       pl.BlockSpec((8, D), lambda i: (0, 0))],
            out_shape=[jax.ShapeDtypeStruct((EH, D), jnp.bfloat16),
                       jax.ShapeDtypeStruct((EH, D), jnp.bfloat16),
                       jax.ShapeDtypeStruct((8, D), f32)],
        )(g1, g1, ea_b, w1e, b1,
          p['cart_Wg2'], row2(p['cart_bg2']),
          p['cart_Wa2'], row2(p['cart_ba2']))

    ga, ma, sta = cart_half(0)
    gb, mb, stb = cart_half(1)

    def cart_tail(h, g_arr, m_arr):
        lo = h * EH
        sm = pl.pallas_call(
            _edge2_body,
            grid=(GEh,),
            in_specs=[espec(D), espec(D), _FULL((8, D)), _FULL((8, D)),
                      _FULL((1, D)), _FULL((1, D))],
            out_specs=pl.BlockSpec((2, BE, D), lambda i: (0, i, 0)),
            out_shape=jax.ShapeDtypeStruct((2, EH, D), f32),
        )(g_arr, m_arr, sta, stb,
          row2(p['cart_bne_g']), row2(p['cart_bne_b']))
        dh = lax.dynamic_slice(dst, (lo,), (EH,))
        return _sc_scatter(sm.reshape(2 * EH, D),
                           jnp.concatenate([dh, dh]), 200)

    nd_a = cart_tail(0, ga, ma)
    nd_b = cart_tail(1, gb, mb)

    h, t2, xr = pl.pallas_call(
        _n2_body,
        grid=(),
        in_specs=[_FULL((N, D)), _FULL((2, N, D)),
                  _FULL((N, 1)),
                  _FULL((1, D)), _FULL((1, D)),
                  _FULL((1, D)), _FULL((1, D)), _FULL((1, D)),
                  _FULL((D, D)), _FULL((1, D)),
                  _FULL((D, D)), _FULL((1, D)),
                  _FULL((D, D)), _FULL((1, D)),
                  _FULL((D, D)), _FULL((1, D))],
        out_specs=[_FULL((N, D)), _FULL((N, 2 * D)), _FULL((N, D))],
        out_shape=[jax.ShapeDtypeStruct((N, D), f32),
                   jax.ShapeDtypeStruct((N, 2 * D), jnp.int32),
                   jax.ShapeDtypeStruct((N, D), f32)],
    )(x, nd_a + nd_b, b2d,
      row2(p['cart_bnx_g']), row2(p['cart_bnx_b']),
      row2(p['gnc_w']), row2(p['gnc_b']), row2(p['gnc_ms']),
      p['mat_Wq'], row2(p['mat_bq']), p['mat_Wk'], row2(p['mat_bk']),
      p['mat_Wv'], row2(p['mat_bv']), p['mat_Wskip'], row2(p['mat_bskip']))

    def mat_half(h):
        lo = h * EH
        idx2 = jnp.concatenate([lax.dynamic_slice(dst, (lo,), (EH,)),
                                lax.dynamic_slice(src, (lo,), (EH,))])
        g2 = _sc_gather(t2, idx2, 2 * D, 200)
        return pl.pallas_call(
            _edge3_body,
            grid=(GEh,),
            in_specs=[espec(2 * D), espec_at(2 * D, GEh), espec_at(D, h * GEh),
                      _FULL((D, D)),
                      _FULL((D, 3 * D)), _FULL((D, 3 * D)), _FULL((D, 3 * D)),
                      _FULL((1, 3 * D)), _FULL((1, 3 * D)), _FULL((1, 3 * D)),
                      _FULL((3 * D, D)), _FULL((1, D)),
                      _FULL((1, D)), _FULL((1, D))],
            out_specs=espec(D),
            out_shape=jax.ShapeDtypeStruct((EH, D), f32),
        )(g2, g2, ea_b, p['mat_We'],
          p['mat_Wmu'][:D], p['mat_Wmu'][D:2 * D], p['mat_Wmu'][2 * D:],
          row2(p['mat_bmu']), row2(p['mat_lna_g']), row2(p['mat_lna_b']),
          p['mat_Wml'], row2(p['mat_bml']),
          row2(p['mat_lnm_g']), row2(p['mat_lnm_b']))

    msg = jnp.concatenate([mat_half(0), mat_half(1)], axis=0)

    mo = _sc_scatter(msg, dst, 200)

    return pl.pallas_call(
        _n4_body,
        grid=(),
        in_specs=[_FULL((N, D)), _FULL((2, N, D)), _FULL((N, D)),
                  _FULL((N, 1)), _FULL((D, 1)), _FULL((D, 1)),
                  _FULL((1, D)), _FULL((1, D)), _FULL((1, D))],
        out_specs=_FULL((N, D)),
        out_shape=jax.ShapeDtypeStruct((N, D), f32),
    )(h, mo, xr, b2d, wbo, wbx,
      row2(p['gnm_w']), row2(p['gnm_b']), row2(p['gnm_ms']))
